# baseline (speedup 1.0000x reference)
import functools

import jax
import jax.numpy as jnp
from jax import lax
from jax.experimental import pallas as pl
from jax.experimental.pallas import tpu as pltpu
from jax.experimental.pallas import tpu_sc as plsc

HIDDEN = 1024
FFN = 4096
E = 8
T = 4096
A = 2 * T
BLK = 256
P = A + E * BLK
NB = P // BLK
NBP = 48
TOK_BLK = 512

NTILE = 16
CH = A // NTILE
NG = CH // 16
PCOLS = 128
PROWS = P // PCOLS

GT = 32
GSL = P // GT
GCH = 80
GD = HIDDEN // 2
TPT = T // GT
CC = 16


def _router_body(x_ref, wr_ref, eidx_ref, ew_ref):
    xb = x_ref[...]
    logits = lax.dot_general(xb, wr_ref[...], (((1,), (1,)), ((), ())),
                             preferred_element_type=jnp.float32)
    m = jnp.max(logits, axis=1, keepdims=True)
    ex = jnp.exp(logits - m)
    probs = ex / jnp.sum(ex, axis=1, keepdims=True)

    iota = lax.broadcasted_iota(jnp.int32, probs.shape, 1)
    m1 = jnp.max(probs, axis=1, keepdims=True)
    idx1 = jnp.min(jnp.where(probs == m1, iota, E), axis=1, keepdims=True)
    p2 = jnp.where(iota == idx1, -jnp.inf, probs)
    m2 = jnp.max(p2, axis=1, keepdims=True)
    idx2 = jnp.min(jnp.where(p2 == m2, iota, E), axis=1, keepdims=True)
    eidx_ref[...] = jnp.concatenate([idx1.T, idx2.T], axis=0)
    ew_ref[...] = jnp.concatenate([m1.T, m2.T], axis=0)


def _dispatch_body(eidx_hbm, ew_hbm, gidx_hbm, gw_hbm, pos_hbm, bexp_hbm,
                   ids_v, ws_v, pos_v, gidx_v, gw_v, vec_v, all_v, cur_v,
                   bexp_v, rowi_v, sh_cnt, sh_gidx, sh_gw):
    wid = lax.axis_index("s")
    base = wid * CH
    lane = lax.iota(jnp.int32, 16)
    z16i = jnp.zeros((16,), jnp.int32)

    pltpu.sync_copy(eidx_hbm.at[pl.ds(base, CH)], ids_v)
    pltpu.sync_copy(ew_hbm.at[pl.ds(base, CH)], ws_v)

    def _zrow(i, c):
        for k in range(PCOLS // 16):
            gidx_v[i, pl.ds(k * 16, 16)] = z16i
            gw_v[i, pl.ds(k * 16, 16)] = z16i
        return c
    lax.fori_loop(0, PROWS, _zrow, 0)

    def _riota(j, c):
        rowi_v[pl.ds(j * 16, 16)] = j * 16 + lane
        return c
    lax.fori_loop(0, PROWS // 16, _riota, 0)

    def _hist(g, cnt):
        ids16 = ids_v[pl.ds(g * 16, 16)]
        for e in range(E):
            c = jnp.sum((ids16 == e).astype(jnp.int32))
            cnt = cnt + jnp.where(lane == e, c, 0)
        return cnt
    cnt = lax.fori_loop(0, NG, _hist, z16i)
    vec_v[...] = cnt
    pltpu.sync_copy(vec_v, sh_cnt.at[pl.ds(wid * 16, 16)])

    @pl.when(wid == 0)
    def _():
        pltpu.sync_copy(gidx_v, sh_gidx)
        pltpu.sync_copy(gw_v, sh_gw)

    plsc.subcore_barrier()

    pltpu.sync_copy(sh_cnt, all_v)
    tot = z16i
    pre = z16i
    for w in range(NTILE):
        row = all_v[pl.ds(w * 16, 16)]
        tot = tot + row
        pre = pre + jnp.where(w < wid, row, z16i)
    padded = ((tot + (BLK - 1)) >> 8) << 8
    inc = plsc.cumsum(padded)
    off = inc - padded
    cur_v[...] = off + pre

    @pl.when(wid == 0)
    def _():
        binc = plsc.cumsum(padded >> 8)
        for c in range(NBP // 16):
            bv = lane + c * 16
            acc = z16i
            for e in range(E):
                s_e = jnp.sum(jnp.where(lane == e, binc, 0))
                acc = acc + (bv >= s_e).astype(jnp.int32)
            bexp_v[pl.ds(c * 16, 16)] = jnp.minimum(acc, E - 1)
        pltpu.sync_copy(bexp_v, bexp_hbm)

    def _assign(g, c):
        ids16 = ids_v[pl.ds(g * 16, 16)]
        ws16 = ws_v[pl.ds(g * 16, 16)]
        tok16 = (base + g * 16 + lane) & (T - 1)
        curv = plsc.load_gather(cur_v, [ids16])
        rank = z16i
        upd = z16i
        for e in range(E):
            oh = ids16 == e
            ohi = oh.astype(jnp.int32)
            cs = plsc.cumsum(ohi)
            rank = rank + jnp.where(oh, cs - 1, z16i)
            upd = upd + jnp.where(lane == e, jnp.sum(ohi), 0)
        dest = curv + rank
        cur_v[...] = cur_v[...] + upd
        plsc.store_scatter(gidx_v, [dest >> 7, dest & (PCOLS - 1)], tok16)
        plsc.store_scatter(gw_v, [dest >> 7, dest & (PCOLS - 1)],
                           plsc.bitcast(ws16, jnp.int32))
        pos_v[pl.ds(g * 16, 16)] = dest
        return c
    lax.fori_loop(0, NG, _assign, 0)

    pltpu.sync_copy(pos_v, pos_hbm.at[pl.ds(base, CH)])

    plsc.subcore_barrier()
    pltpu.sync_copy(gidx_v, sh_gidx.at[rowi_v], add=True)
    pltpu.sync_copy(gw_v, sh_gw.at[rowi_v], add=True)
    plsc.subcore_barrier()

    @pl.when(wid < PROWS // 8)
    def _():
        pltpu.sync_copy(sh_gidx.at[pl.ds(wid * 8, 8)],
                        gidx_hbm.at[pl.ds(wid * 8, 8)])
        pltpu.sync_copy(sh_gw.at[pl.ds(wid * 8, 8)],
                        gw_hbm.at[pl.ds(wid * 8, 8)])


def _gather_body(flat_hbm, gidx_hbm, xg_hbm, idx_v, rows0_v, rows1_v, sem0,
                 sem1):
    wid = lax.axis_index("s") * 2 + lax.axis_index("c")
    base = wid * GSL
    nch = GSL // GCH
    bufs = (rows0_v, rows1_v)
    sems = (sem0, sem1)
    pltpu.sync_copy(gidx_hbm.at[pl.ds(base, GSL)], idx_v)
    cps = [None, None]
    cps[0] = pltpu.async_copy(flat_hbm.at[idx_v.at[pl.ds(0, GCH)]],
                              bufs[0], sems[0])
    for j in range(nch):
        if j + 1 < nch:
            cps[(j + 1) % 2] = pltpu.async_copy(
                flat_hbm.at[idx_v.at[pl.ds((j + 1) * GCH, GCH)]],
                bufs[(j + 1) % 2], sems[(j + 1) % 2])
        cps[j % 2].wait()
        pltpu.sync_copy(bufs[j % 2], xg_hbm.at[pl.ds(base + j * GCH, GCH)])


def _ffn1_body(bexp_ref, xg_ref, w1_ref, b1_ref, h_ref):
    xb = xg_ref[...]
    h = lax.dot_general(xb, w1_ref[0], (((1,), (1,)), ((), ())),
                        preferred_element_type=jnp.float32)
    h = h + b1_ref[0]
    h_ref[...] = (h * jax.nn.sigmoid(h)).astype(jnp.bfloat16)


def _ffn2_body(bexp_ref, h_ref, w2_ref, b2_ref, gw_ref, og_ref):
    o = lax.dot_general(h_ref[...], w2_ref[0], (((1,), (1,)), ((), ())),
                        preferred_element_type=jnp.float32)
    o = o + b2_ref[0]
    og_ref[...] = o * gw_ref[0, 0][:, None]


def _combine_body(og_hbm, pos_hbm, out_hbm, idx0_v, idx1_v, bufa0_v, bufa1_v,
                  bufb0_v, bufb1_v, sem0, sem1):
    wid = lax.axis_index("s") * 2 + lax.axis_index("c")
    tbase = wid * TPT
    nch = TPT // CC
    bufs = ((bufa0_v, bufa1_v), (bufb0_v, bufb1_v))
    sems = (sem0, sem1)
    pltpu.sync_copy(pos_hbm.at[pl.ds(tbase, TPT)], idx0_v)
    pltpu.sync_copy(pos_hbm.at[pl.ds(T + tbase, TPT)], idx1_v)

    def _start(j):
        b0, b1 = bufs[j % 2]
        s = sems[j % 2]
        c0 = pltpu.async_copy(og_hbm.at[idx0_v.at[pl.ds(j * CC, CC)]], b0, s)
        c1 = pltpu.async_copy(og_hbm.at[idx1_v.at[pl.ds(j * CC, CC)]], b1, s)
        return (c0, c1)

    cps = [None, None]
    cps[0] = _start(0)
    for j in range(nch):
        if j + 1 < nch:
            cps[(j + 1) % 2] = _start(j + 1)
        cps[j % 2][0].wait()
        cps[j % 2][1].wait()
        b0, b1 = bufs[j % 2]

        def _row(i, c):
            for k in range(HIDDEN // 16):
                s = pl.ds(k * 16, 16)
                b0[i, s] = b0[i, s] + b1[i, s]
            return c
        lax.fori_loop(0, CC, _row, 0)
        pltpu.sync_copy(b0, out_hbm.at[pl.ds(tbase + j * CC, CC)])


def _run_router(flat, Wr):
    return pl.pallas_call(
        _router_body,
        grid=(T // TOK_BLK,),
        in_specs=[
            pl.BlockSpec((TOK_BLK, HIDDEN), lambda t: (t, 0)),
            pl.BlockSpec((E, HIDDEN), lambda t: (0, 0)),
        ],
        out_specs=[
            pl.BlockSpec((2, TOK_BLK), lambda t: (0, t)),
            pl.BlockSpec((2, TOK_BLK), lambda t: (0, t)),
        ],
        out_shape=[
            jax.ShapeDtypeStruct((2, T), jnp.int32),
            jax.ShapeDtypeStruct((2, T), jnp.float32),
        ],
    )(flat, Wr)


def _run_dispatch(eidx, ew):
    mesh1 = plsc.VectorSubcoreMesh(core_axis_name="c", subcore_axis_name="s",
                                   num_cores=1, num_subcores=NTILE)
    dispatch = functools.partial(
        pl.kernel,
        out_type=[
            jax.ShapeDtypeStruct((PROWS, PCOLS), jnp.int32),
            jax.ShapeDtypeStruct((PROWS, PCOLS), jnp.int32),
            jax.ShapeDtypeStruct((A,), jnp.int32),
            jax.ShapeDtypeStruct((NBP,), jnp.int32),
        ],
        mesh=mesh1,
        scratch_types=[
            pltpu.VMEM((CH,), jnp.int32),
            pltpu.VMEM((CH,), jnp.float32),
            pltpu.VMEM((CH,), jnp.int32),
            pltpu.VMEM((PROWS, PCOLS), jnp.int32),
            pltpu.VMEM((PROWS, PCOLS), jnp.int32),
            pltpu.VMEM((16,), jnp.int32),
            pltpu.VMEM((NTILE * 16,), jnp.int32),
            pltpu.VMEM((16,), jnp.int32),
            pltpu.VMEM((NBP,), jnp.int32),
            pltpu.VMEM((PROWS,), jnp.int32),
            pltpu.VMEM_SHARED((NTILE * 16,), jnp.int32),
            pltpu.VMEM_SHARED((PROWS, PCOLS), jnp.int32),
            pltpu.VMEM_SHARED((PROWS, PCOLS), jnp.int32),
        ],
        compiler_params=pltpu.CompilerParams(needs_layout_passes=False),
    )(_dispatch_body)
    gidx2, gw2i, pos, bexp = dispatch(eidx.reshape(A), ew.reshape(A))
    gidx = gidx2.reshape(P)
    gw2 = lax.bitcast_convert_type(gw2i, jnp.float32)
    return gidx, gw2, pos, bexp


def _run_gather(flat_pack, gidx):
    mesh2 = plsc.VectorSubcoreMesh(core_axis_name="c", subcore_axis_name="s",
                                   num_cores=2, num_subcores=NTILE)
    gather = functools.partial(
        pl.kernel,
        out_type=jax.ShapeDtypeStruct((P, GD), jnp.int32),
        mesh=mesh2,
        scratch_types=[
            pltpu.VMEM((GSL,), jnp.int32),
            pltpu.VMEM((GCH, GD), jnp.int32),
            pltpu.VMEM((GCH, GD), jnp.int32),
            pltpu.SemaphoreType.DMA,
            pltpu.SemaphoreType.DMA,
        ],
        compiler_params=pltpu.CompilerParams(needs_layout_passes=False),
    )(_gather_body)
    return gather(flat_pack, gidx)


def _run_ffn(xgb, W1, b1, W2, b2, gw2, bexp):
    h_all = pl.pallas_call(
        _ffn1_body,
        grid_spec=pltpu.PrefetchScalarGridSpec(
            num_scalar_prefetch=1,
            grid=(NB,),
            in_specs=[
                pl.BlockSpec((BLK, HIDDEN), lambda b, be: (b, 0)),
                pl.BlockSpec((1, FFN, HIDDEN), lambda b, be: (be[b], 0, 0)),
                pl.BlockSpec((1, 1, FFN), lambda b, be: (be[b], 0, 0)),
            ],
            out_specs=pl.BlockSpec((BLK, FFN), lambda b, be: (b, 0)),
        ),
        out_shape=jax.ShapeDtypeStruct((P, FFN), jnp.bfloat16),
    )(bexp, xgb, W1.astype(jnp.bfloat16), b1.reshape(E, 1, FFN))

    og = pl.pallas_call(
        _ffn2_body,
        grid_spec=pltpu.PrefetchScalarGridSpec(
            num_scalar_prefetch=1,
            grid=(NB,),
            in_specs=[
                pl.BlockSpec((BLK, FFN), lambda b, be: (b, 0)),
                pl.BlockSpec((1, HIDDEN, FFN), lambda b, be: (be[b], 0, 0)),
                pl.BlockSpec((1, 1, HIDDEN), lambda b, be: (be[b], 0, 0)),
                pl.BlockSpec((1, 1, BLK), lambda b, be: (b, 0, 0)),
            ],
            out_specs=pl.BlockSpec((BLK, HIDDEN), lambda b, be: (b, 0)),
        ),
        out_shape=jax.ShapeDtypeStruct((P, HIDDEN), jnp.float32),
    )(bexp, h_all, W2.astype(jnp.bfloat16), b2.reshape(E, 1, HIDDEN),
      gw2.reshape(NB, 1, BLK))
    return og


def _run_combine(og, pos):
    mesh2 = plsc.VectorSubcoreMesh(core_axis_name="c", subcore_axis_name="s",
                                   num_cores=2, num_subcores=NTILE)
    combine = functools.partial(
        pl.kernel,
        out_type=jax.ShapeDtypeStruct((T, HIDDEN), jnp.float32),
        mesh=mesh2,
        scratch_types=[
            pltpu.VMEM((TPT,), jnp.int32),
            pltpu.VMEM((TPT,), jnp.int32),
            pltpu.VMEM((CC, HIDDEN), jnp.float32),
            pltpu.VMEM((CC, HIDDEN), jnp.float32),
            pltpu.VMEM((CC, HIDDEN), jnp.float32),
            pltpu.VMEM((CC, HIDDEN), jnp.float32),
            pltpu.SemaphoreType.DMA,
            pltpu.SemaphoreType.DMA,
        ],
        compiler_params=pltpu.CompilerParams(needs_layout_passes=False),
    )(_combine_body)
    return combine(og, pos)


def kernel(x, Wr, W1, b1, W2, b2):
    batch, seq, hidden = x.shape
    flat = x.reshape(T, hidden)
    eidx, ew = _run_router(flat, Wr)
    gidx, gw2, pos, bexp = _run_dispatch(eidx, ew)
    flat_pack = lax.bitcast_convert_type(
        flat.astype(jnp.bfloat16).reshape(T, GD, 2), jnp.int32)
    xgp = _run_gather(flat_pack, gidx)
    xgb = lax.bitcast_convert_type(xgp, jnp.bfloat16).reshape(P, HIDDEN)
    og = _run_ffn(xgb, W1, b1, W2, b2, gw2, bexp)
    out = _run_combine(og, pos)
    return out.reshape(batch, seq, hidden)


# speedup vs baseline: 1.1049x; 1.1049x over previous
import functools

import jax
import jax.numpy as jnp
from jax import lax
from jax.experimental import pallas as pl
from jax.experimental.pallas import tpu as pltpu
from jax.experimental.pallas import tpu_sc as plsc

HIDDEN = 1024
FFN = 4096
E = 8
T = 4096
A = 2 * T
BLK = 256
P = A + E * BLK
NB = P // BLK
NBP = 48
TOK_BLK = 512

NTILE = 16
CH = A // NTILE
NG = CH // 16
PCOLS = 128
PROWS = P // PCOLS

GT = 32
GSL = P // GT
GCH = 80
GD = HIDDEN // 2
TPT = T // GT
CC = 16


def _router_body(x_ref, wr_ref, eidx_ref, ew_ref):
    xb = x_ref[...]
    logits = lax.dot_general(xb, wr_ref[...], (((1,), (1,)), ((), ())),
                             preferred_element_type=jnp.float32)
    m = jnp.max(logits, axis=1, keepdims=True)
    ex = jnp.exp(logits - m)
    probs = ex / jnp.sum(ex, axis=1, keepdims=True)

    iota = lax.broadcasted_iota(jnp.int32, probs.shape, 1)
    m1 = jnp.max(probs, axis=1, keepdims=True)
    idx1 = jnp.min(jnp.where(probs == m1, iota, E), axis=1, keepdims=True)
    p2 = jnp.where(iota == idx1, -jnp.inf, probs)
    m2 = jnp.max(p2, axis=1, keepdims=True)
    idx2 = jnp.min(jnp.where(p2 == m2, iota, E), axis=1, keepdims=True)
    eidx_ref[...] = jnp.concatenate([idx1.T, idx2.T], axis=0)
    ew_ref[...] = jnp.concatenate([m1.T, m2.T], axis=0)


def _dispatch_body(eidx_hbm, ew_hbm, gidx_hbm, gw_hbm, pos_hbm, bexp_hbm,
                   ids_v, ws_v, pos_v, gidx_v, gw_v, vec_v, all_v, cur_v,
                   bexp_v, rowi_v, sh_cnt, sh_gidx, sh_gw):
    wid = lax.axis_index("s")
    base = wid * CH
    lane = lax.iota(jnp.int32, 16)
    z16i = jnp.zeros((16,), jnp.int32)

    pltpu.sync_copy(eidx_hbm.at[pl.ds(base, CH)], ids_v)
    pltpu.sync_copy(ew_hbm.at[pl.ds(base, CH)], ws_v)

    def _zrow(i, c):
        for k in range(PCOLS // 16):
            gidx_v[i, pl.ds(k * 16, 16)] = z16i
            gw_v[i, pl.ds(k * 16, 16)] = z16i
        return c
    lax.fori_loop(0, PROWS, _zrow, 0)

    def _riota(j, c):
        rowi_v[pl.ds(j * 16, 16)] = j * 16 + lane
        return c
    lax.fori_loop(0, PROWS // 16, _riota, 0)

    def _hist(g, cnt):
        ids16 = ids_v[pl.ds(g * 16, 16)]
        for e in range(E):
            c = jnp.sum((ids16 == e).astype(jnp.int32))
            cnt = cnt + jnp.where(lane == e, c, 0)
        return cnt
    cnt = lax.fori_loop(0, NG, _hist, z16i)
    vec_v[...] = cnt
    pltpu.sync_copy(vec_v, sh_cnt.at[pl.ds(wid * 16, 16)])

    @pl.when(wid == 0)
    def _():
        pltpu.sync_copy(gidx_v, sh_gidx)
        pltpu.sync_copy(gw_v, sh_gw)

    plsc.subcore_barrier()

    pltpu.sync_copy(sh_cnt, all_v)
    tot = z16i
    pre = z16i
    for w in range(NTILE):
        row = all_v[pl.ds(w * 16, 16)]
        tot = tot + row
        pre = pre + jnp.where(w < wid, row, z16i)
    padded = ((tot + (BLK - 1)) >> 8) << 8
    inc = plsc.cumsum(padded)
    off = inc - padded
    cur_v[...] = off + pre

    @pl.when(wid == 0)
    def _():
        binc = plsc.cumsum(padded >> 8)
        nbu = jnp.sum(jnp.where(lane == E - 1, binc, 0))
        for c in range(NBP // 16):
            bv = lane + c * 16
            acc = z16i
            for e in range(E):
                s_e = jnp.sum(jnp.where(lane == e, binc, 0))
                acc = acc + (bv >= s_e).astype(jnp.int32)
            ch = jnp.minimum(acc, E - 1)
            if c * 16 <= NB < (c + 1) * 16:
                ch = jnp.where(lane == NB - c * 16, nbu, ch)
            bexp_v[pl.ds(c * 16, 16)] = ch
        pltpu.sync_copy(bexp_v, bexp_hbm)

    def _assign(g, c):
        ids16 = ids_v[pl.ds(g * 16, 16)]
        ws16 = ws_v[pl.ds(g * 16, 16)]
        tok16 = (base + g * 16 + lane) & (T - 1)
        curv = plsc.load_gather(cur_v, [ids16])
        rank = z16i
        upd = z16i
        for e in range(E):
            oh = ids16 == e
            ohi = oh.astype(jnp.int32)
            cs = plsc.cumsum(ohi)
            rank = rank + jnp.where(oh, cs - 1, z16i)
            upd = upd + jnp.where(lane == e, jnp.sum(ohi), 0)
        dest = curv + rank
        cur_v[...] = cur_v[...] + upd
        plsc.store_scatter(gidx_v, [dest >> 7, dest & (PCOLS - 1)], tok16)
        plsc.store_scatter(gw_v, [dest >> 7, dest & (PCOLS - 1)],
                           plsc.bitcast(ws16, jnp.int32))
        pos_v[pl.ds(g * 16, 16)] = dest
        return c
    lax.fori_loop(0, NG, _assign, 0)

    pltpu.sync_copy(pos_v, pos_hbm.at[pl.ds(base, CH)])

    plsc.subcore_barrier()
    pltpu.sync_copy(gidx_v, sh_gidx.at[rowi_v], add=True)
    pltpu.sync_copy(gw_v, sh_gw.at[rowi_v], add=True)
    plsc.subcore_barrier()

    @pl.when(wid < PROWS // 8)
    def _():
        pltpu.sync_copy(sh_gidx.at[pl.ds(wid * 8, 8)],
                        gidx_hbm.at[pl.ds(wid * 8, 8)])
        pltpu.sync_copy(sh_gw.at[pl.ds(wid * 8, 8)],
                        gw_hbm.at[pl.ds(wid * 8, 8)])


def _gather_body(flat_hbm, gidx_hbm, xg_hbm, idx_v, rows0_v, rows1_v, sem0,
                 sem1):
    wid = lax.axis_index("s") * 2 + lax.axis_index("c")
    base = wid * GSL
    nch = GSL // GCH
    bufs = (rows0_v, rows1_v)
    sems = (sem0, sem1)
    pltpu.sync_copy(gidx_hbm.at[pl.ds(base, GSL)], idx_v)
    cps = [None, None]
    cps[0] = pltpu.async_copy(flat_hbm.at[idx_v.at[pl.ds(0, GCH)]],
                              bufs[0], sems[0])
    for j in range(nch):
        if j + 1 < nch:
            cps[(j + 1) % 2] = pltpu.async_copy(
                flat_hbm.at[idx_v.at[pl.ds((j + 1) * GCH, GCH)]],
                bufs[(j + 1) % 2], sems[(j + 1) % 2])
        cps[j % 2].wait()
        pltpu.sync_copy(bufs[j % 2], xg_hbm.at[pl.ds(base + j * GCH, GCH)])


def _ffn1_body(bexp_ref, xg_ref, w1_ref, b1_ref, h_ref):
    @pl.when(pl.program_id(0) < bexp_ref[NB])
    def _():
        xb = xg_ref[...].astype(jnp.float32)
        h = lax.dot_general(xb, w1_ref[0], (((1,), (1,)), ((), ())),
                            preferred_element_type=jnp.float32)
        h = h + b1_ref[0]
        h_ref[...] = h * jax.nn.sigmoid(h)


def _ffn2_body(bexp_ref, h_ref, w2_ref, b2_ref, gw_ref, og_ref):
    @pl.when(pl.program_id(0) < bexp_ref[NB])
    def _():
        o = lax.dot_general(h_ref[...], w2_ref[0], (((1,), (1,)), ((), ())),
                            preferred_element_type=jnp.float32)
        o = o + b2_ref[0]
        og_ref[...] = o * gw_ref[0, 0][:, None]


def _combine_body(og_hbm, pos_hbm, out_hbm, idx0_v, idx1_v, bufa0_v, bufa1_v,
                  bufb0_v, bufb1_v, sem0, sem1):
    wid = lax.axis_index("s") * 2 + lax.axis_index("c")
    tbase = wid * TPT
    nch = TPT // CC
    bufs = ((bufa0_v, bufa1_v), (bufb0_v, bufb1_v))
    sems = (sem0, sem1)
    pltpu.sync_copy(pos_hbm.at[pl.ds(tbase, TPT)], idx0_v)
    pltpu.sync_copy(pos_hbm.at[pl.ds(T + tbase, TPT)], idx1_v)

    def _start(j):
        b0, b1 = bufs[j % 2]
        s = sems[j % 2]
        c0 = pltpu.async_copy(og_hbm.at[idx0_v.at[pl.ds(j * CC, CC)]], b0, s)
        c1 = pltpu.async_copy(og_hbm.at[idx1_v.at[pl.ds(j * CC, CC)]], b1, s)
        return (c0, c1)

    cps = [None, None]
    cps[0] = _start(0)
    for j in range(nch):
        if j + 1 < nch:
            cps[(j + 1) % 2] = _start(j + 1)
        cps[j % 2][0].wait()
        cps[j % 2][1].wait()
        b0, b1 = bufs[j % 2]

        def _row(i, c):
            for k in range(HIDDEN // 16):
                s = pl.ds(k * 16, 16)
                b0[i, s] = b0[i, s] + b1[i, s]
            return c
        lax.fori_loop(0, CC, _row, 0)
        pltpu.sync_copy(b0, out_hbm.at[pl.ds(tbase + j * CC, CC)])


def _run_router(flat, Wr):
    return pl.pallas_call(
        _router_body,
        grid=(T // TOK_BLK,),
        in_specs=[
            pl.BlockSpec((TOK_BLK, HIDDEN), lambda t: (t, 0)),
            pl.BlockSpec((E, HIDDEN), lambda t: (0, 0)),
        ],
        out_specs=[
            pl.BlockSpec((2, TOK_BLK), lambda t: (0, t)),
            pl.BlockSpec((2, TOK_BLK), lambda t: (0, t)),
        ],
        out_shape=[
            jax.ShapeDtypeStruct((2, T), jnp.int32),
            jax.ShapeDtypeStruct((2, T), jnp.float32),
        ],
    )(flat, Wr)


def _run_dispatch(eidx, ew):
    mesh1 = plsc.VectorSubcoreMesh(core_axis_name="c", subcore_axis_name="s",
                                   num_cores=1, num_subcores=NTILE)
    dispatch = functools.partial(
        pl.kernel,
        out_type=[
            jax.ShapeDtypeStruct((PROWS, PCOLS), jnp.int32),
            jax.ShapeDtypeStruct((PROWS, PCOLS), jnp.int32),
            jax.ShapeDtypeStruct((A,), jnp.int32),
            jax.ShapeDtypeStruct((NBP,), jnp.int32),
        ],
        mesh=mesh1,
        scratch_types=[
            pltpu.VMEM((CH,), jnp.int32),
            pltpu.VMEM((CH,), jnp.float32),
            pltpu.VMEM((CH,), jnp.int32),
            pltpu.VMEM((PROWS, PCOLS), jnp.int32),
            pltpu.VMEM((PROWS, PCOLS), jnp.int32),
            pltpu.VMEM((16,), jnp.int32),
            pltpu.VMEM((NTILE * 16,), jnp.int32),
            pltpu.VMEM((16,), jnp.int32),
            pltpu.VMEM((NBP,), jnp.int32),
            pltpu.VMEM((PROWS,), jnp.int32),
            pltpu.VMEM_SHARED((NTILE * 16,), jnp.int32),
            pltpu.VMEM_SHARED((PROWS, PCOLS), jnp.int32),
            pltpu.VMEM_SHARED((PROWS, PCOLS), jnp.int32),
        ],
        compiler_params=pltpu.CompilerParams(needs_layout_passes=False),
    )(_dispatch_body)
    gidx2, gw2i, pos, bexp = dispatch(eidx.reshape(A), ew.reshape(A))
    gidx = gidx2.reshape(P)
    gw2 = lax.bitcast_convert_type(gw2i, jnp.float32)
    return gidx, gw2, pos, bexp


def _run_gather(flat_pack, gidx):
    mesh2 = plsc.VectorSubcoreMesh(core_axis_name="c", subcore_axis_name="s",
                                   num_cores=2, num_subcores=NTILE)
    gather = functools.partial(
        pl.kernel,
        out_type=jax.ShapeDtypeStruct((P, GD), jnp.int32),
        mesh=mesh2,
        scratch_types=[
            pltpu.VMEM((GSL,), jnp.int32),
            pltpu.VMEM((GCH, GD), jnp.int32),
            pltpu.VMEM((GCH, GD), jnp.int32),
            pltpu.SemaphoreType.DMA,
            pltpu.SemaphoreType.DMA,
        ],
        compiler_params=pltpu.CompilerParams(needs_layout_passes=False),
    )(_gather_body)
    return gather(flat_pack, gidx)


def _run_ffn(xgb, W1, b1, W2, b2, gw2, bexp):
    h_all = pl.pallas_call(
        _ffn1_body,
        grid_spec=pltpu.PrefetchScalarGridSpec(
            num_scalar_prefetch=1,
            grid=(NB,),
            in_specs=[
                pl.BlockSpec((BLK, HIDDEN), lambda b, be: (b, 0)),
                pl.BlockSpec((1, FFN, HIDDEN), lambda b, be: (be[b], 0, 0)),
                pl.BlockSpec((1, 1, FFN), lambda b, be: (be[b], 0, 0)),
            ],
            out_specs=pl.BlockSpec((BLK, FFN), lambda b, be: (b, 0)),
        ),
        out_shape=jax.ShapeDtypeStruct((P, FFN), jnp.float32),
    )(bexp, xgb, W1, b1.reshape(E, 1, FFN))

    og = pl.pallas_call(
        _ffn2_body,
        grid_spec=pltpu.PrefetchScalarGridSpec(
            num_scalar_prefetch=1,
            grid=(NB,),
            in_specs=[
                pl.BlockSpec((BLK, FFN), lambda b, be: (b, 0)),
                pl.BlockSpec((1, HIDDEN, FFN), lambda b, be: (be[b], 0, 0)),
                pl.BlockSpec((1, 1, HIDDEN), lambda b, be: (be[b], 0, 0)),
                pl.BlockSpec((1, 1, BLK), lambda b, be: (b, 0, 0)),
            ],
            out_specs=pl.BlockSpec((BLK, HIDDEN), lambda b, be: (b, 0)),
        ),
        out_shape=jax.ShapeDtypeStruct((P, HIDDEN), jnp.float32),
    )(bexp, h_all, W2, b2.reshape(E, 1, HIDDEN), gw2.reshape(NB, 1, BLK))
    return og


def _run_combine(og, pos):
    mesh2 = plsc.VectorSubcoreMesh(core_axis_name="c", subcore_axis_name="s",
                                   num_cores=2, num_subcores=NTILE)
    combine = functools.partial(
        pl.kernel,
        out_type=jax.ShapeDtypeStruct((T, HIDDEN), jnp.float32),
        mesh=mesh2,
        scratch_types=[
            pltpu.VMEM((TPT,), jnp.int32),
            pltpu.VMEM((TPT,), jnp.int32),
            pltpu.VMEM((CC, HIDDEN), jnp.float32),
            pltpu.VMEM((CC, HIDDEN), jnp.float32),
            pltpu.VMEM((CC, HIDDEN), jnp.float32),
            pltpu.VMEM((CC, HIDDEN), jnp.float32),
            pltpu.SemaphoreType.DMA,
            pltpu.SemaphoreType.DMA,
        ],
        compiler_params=pltpu.CompilerParams(needs_layout_passes=False),
    )(_combine_body)
    return combine(og, pos)


def kernel(x, Wr, W1, b1, W2, b2):
    batch, seq, hidden = x.shape
    flat = x.reshape(T, hidden)
    eidx, ew = _run_router(flat, Wr)
    gidx, gw2, pos, bexp = _run_dispatch(eidx, ew)
    flat_pack = lax.bitcast_convert_type(
        flat.astype(jnp.bfloat16).reshape(T, GD, 2), jnp.int32)
    xgp = _run_gather(flat_pack, gidx)
    xgb = lax.bitcast_convert_type(xgp, jnp.bfloat16).reshape(P, HIDDEN)
    og = _run_ffn(xgb, W1, b1, W2, b2, gw2, bexp)
    out = _run_combine(og, pos)
    return out.reshape(batch, seq, hidden)


# speedup vs baseline: 1.7000x; 1.5387x over previous
import functools

import jax
import jax.numpy as jnp
from jax import lax
from jax.experimental import pallas as pl
from jax.experimental.pallas import tpu as pltpu
from jax.experimental.pallas import tpu_sc as plsc

HIDDEN = 1024
FFN = 4096
E = 8
T = 4096
A = 2 * T
BLK = 256
P = A + E * BLK
NB = P // BLK
NBP = 48
TOK_BLK = 512

NTILE = 16
CH = A // NTILE
NG = CH // 16
PCOLS = 128
PROWS = P // PCOLS

GT = 32
GSL = P // GT
GCH = 40
GD = HIDDEN // 2
TPT = T // GT
CC = 16


def _router_body(x_ref, wr_ref, eidx_ref, ew_ref):
    xb = x_ref[...]
    logits = lax.dot_general(xb, wr_ref[...], (((1,), (1,)), ((), ())),
                             preferred_element_type=jnp.float32)
    m = jnp.max(logits, axis=1, keepdims=True)
    ex = jnp.exp(logits - m)
    probs = ex / jnp.sum(ex, axis=1, keepdims=True)

    iota = lax.broadcasted_iota(jnp.int32, probs.shape, 1)
    m1 = jnp.max(probs, axis=1, keepdims=True)
    idx1 = jnp.min(jnp.where(probs == m1, iota, E), axis=1, keepdims=True)
    p2 = jnp.where(iota == idx1, -jnp.inf, probs)
    m2 = jnp.max(p2, axis=1, keepdims=True)
    idx2 = jnp.min(jnp.where(p2 == m2, iota, E), axis=1, keepdims=True)
    eidx_ref[...] = jnp.concatenate([idx1.T, idx2.T], axis=0)
    ew_ref[...] = jnp.concatenate([m1.T, m2.T], axis=0)


def _dispatch_body(eidx_hbm, ew_hbm, gidx_hbm, gw_hbm, pos_hbm, bexp_hbm,
                   ids_v, ws_v, pos_v, gidx_v, gw_v, vec_v, all_v, cur_v,
                   bexp_v, rowi_v, sh_cnt, sh_gidx, sh_gw):
    wid = lax.axis_index("s")
    base = wid * CH
    lane = lax.iota(jnp.int32, 16)
    z16i = jnp.zeros((16,), jnp.int32)

    pltpu.sync_copy(eidx_hbm.at[pl.ds(base, CH)], ids_v)
    pltpu.sync_copy(ew_hbm.at[pl.ds(base, CH)], ws_v)

    def _zrow(i, c):
        for k in range(PCOLS // 16):
            gidx_v[i, pl.ds(k * 16, 16)] = z16i
            gw_v[i, pl.ds(k * 16, 16)] = z16i
        return c
    lax.fori_loop(0, PROWS, _zrow, 0)

    def _riota(j, c):
        rowi_v[pl.ds(j * 16, 16)] = j * 16 + lane
        return c
    lax.fori_loop(0, PROWS // 16, _riota, 0)

    def _hist(g, cnt):
        ids16 = ids_v[pl.ds(g * 16, 16)]
        for e in range(E):
            c = jnp.sum((ids16 == e).astype(jnp.int32))
            cnt = cnt + jnp.where(lane == e, c, 0)
        return cnt
    cnt = lax.fori_loop(0, NG, _hist, z16i)
    vec_v[...] = cnt
    pltpu.sync_copy(vec_v, sh_cnt.at[pl.ds(wid * 16, 16)])

    @pl.when(wid == 0)
    def _():
        pltpu.sync_copy(gidx_v, sh_gidx)
        pltpu.sync_copy(gw_v, sh_gw)

    plsc.subcore_barrier()

    pltpu.sync_copy(sh_cnt, all_v)
    tot = z16i
    pre = z16i
    for w in range(NTILE):
        row = all_v[pl.ds(w * 16, 16)]
        tot = tot + row
        pre = pre + jnp.where(w < wid, row, z16i)
    padded = ((tot + (BLK - 1)) >> 8) << 8
    inc = plsc.cumsum(padded)
    off = inc - padded
    cur_v[...] = off + pre

    @pl.when(wid == 0)
    def _():
        binc = plsc.cumsum(padded >> 8)
        nbu = jnp.sum(jnp.where(lane == E - 1, binc, 0))
        for c in range(NBP // 16):
            bv = lane + c * 16
            acc = z16i
            for e in range(E):
                s_e = jnp.sum(jnp.where(lane == e, binc, 0))
                acc = acc + (bv >= s_e).astype(jnp.int32)
            ch = jnp.minimum(acc, E - 1)
            if c * 16 <= NB < (c + 1) * 16:
                ch = jnp.where(lane == NB - c * 16, nbu, ch)
            bexp_v[pl.ds(c * 16, 16)] = ch
        pltpu.sync_copy(bexp_v, bexp_hbm)

    def _assign(g, c):
        ids16 = ids_v[pl.ds(g * 16, 16)]
        ws16 = ws_v[pl.ds(g * 16, 16)]
        tok16 = (base + g * 16 + lane) & (T - 1)
        curv = plsc.load_gather(cur_v, [ids16])
        rank = z16i
        upd = z16i
        for e in range(E):
            oh = ids16 == e
            ohi = oh.astype(jnp.int32)
            cs = plsc.cumsum(ohi)
            rank = rank + jnp.where(oh, cs - 1, z16i)
            upd = upd + jnp.where(lane == e, jnp.sum(ohi), 0)
        dest = curv + rank
        cur_v[...] = cur_v[...] + upd
        plsc.store_scatter(gidx_v, [dest >> 7, dest & (PCOLS - 1)], tok16)
        plsc.store_scatter(gw_v, [dest >> 7, dest & (PCOLS - 1)],
                           plsc.bitcast(ws16, jnp.int32))
        pos_v[pl.ds(g * 16, 16)] = dest
        return c
    lax.fori_loop(0, NG, _assign, 0)

    pltpu.sync_copy(pos_v, pos_hbm.at[pl.ds(base, CH)])

    plsc.subcore_barrier()
    pltpu.sync_copy(gidx_v, sh_gidx.at[rowi_v], add=True)
    pltpu.sync_copy(gw_v, sh_gw.at[rowi_v], add=True)
    plsc.subcore_barrier()

    @pl.when(wid < PROWS // 8)
    def _():
        pltpu.sync_copy(sh_gidx.at[pl.ds(wid * 8, 8)],
                        gidx_hbm.at[pl.ds(wid * 8, 8)])
        pltpu.sync_copy(sh_gw.at[pl.ds(wid * 8, 8)],
                        gw_hbm.at[pl.ds(wid * 8, 8)])


def _gather_body(flat_hbm, gidx_hbm, xg_hbm, idx_v, rows0_v, rows1_v, sem0,
                 sem1):
    wid = lax.axis_index("s") * 2 + lax.axis_index("c")
    base = wid * GSL
    nch = GSL // GCH
    bufs = (rows0_v, rows1_v)
    sems = (sem0, sem1)
    pltpu.sync_copy(gidx_hbm.at[pl.ds(base, GSL)], idx_v)
    cps = [None, None]
    cps[0] = pltpu.async_copy(flat_hbm.at[idx_v.at[pl.ds(0, GCH)]],
                              bufs[0], sems[0])
    for j in range(nch):
        if j + 1 < nch:
            cps[(j + 1) % 2] = pltpu.async_copy(
                flat_hbm.at[idx_v.at[pl.ds((j + 1) * GCH, GCH)]],
                bufs[(j + 1) % 2], sems[(j + 1) % 2])
        cps[j % 2].wait()
        pltpu.sync_copy(bufs[j % 2], xg_hbm.at[pl.ds(base + j * GCH, GCH)])


def _ffn1_body(bexp_ref, xg_ref, w1_ref, b1_ref, h_ref):
    @pl.when(pl.program_id(0) < bexp_ref[NB])
    def _():
        xb = xg_ref[...]
        h = lax.dot_general(xb, w1_ref[0], (((1,), (1,)), ((), ())),
                            preferred_element_type=jnp.float32)
        h = h + b1_ref[0]
        h_ref[...] = h * jax.nn.sigmoid(h)


def _ffn2_body(bexp_ref, h_ref, w2_ref, b2_ref, gw_ref, og_ref):
    @pl.when(pl.program_id(0) < bexp_ref[NB])
    def _():
        o = lax.dot_general(h_ref[...], w2_ref[0], (((1,), (1,)), ((), ())),
                            preferred_element_type=jnp.float32)
        o = o + b2_ref[0]
        og_ref[...] = o * gw_ref[0, 0][:, None]


def _combine_body(og_hbm, pos_hbm, out_hbm, idx0_v, idx1_v, bufa0_v, bufa1_v,
                  bufb0_v, bufb1_v, sem0, sem1):
    wid = lax.axis_index("s") * 2 + lax.axis_index("c")
    tbase = wid * TPT
    nch = TPT // CC
    bufs = ((bufa0_v, bufa1_v), (bufb0_v, bufb1_v))
    sems = (sem0, sem1)
    pltpu.sync_copy(pos_hbm.at[pl.ds(tbase, TPT)], idx0_v)
    pltpu.sync_copy(pos_hbm.at[pl.ds(T + tbase, TPT)], idx1_v)

    def _start(j):
        b0, b1 = bufs[j % 2]
        s = sems[j % 2]
        c0 = pltpu.async_copy(og_hbm.at[idx0_v.at[pl.ds(j * CC, CC)]], b0, s)
        c1 = pltpu.async_copy(og_hbm.at[idx1_v.at[pl.ds(j * CC, CC)]], b1, s)
        return (c0, c1)

    cps = [None, None]
    cps[0] = _start(0)
    for j in range(nch):
        if j + 1 < nch:
            cps[(j + 1) % 2] = _start(j + 1)
        cps[j % 2][0].wait()
        cps[j % 2][1].wait()
        b0, b1 = bufs[j % 2]

        def _row(i, c):
            for k in range(HIDDEN // 16):
                s = pl.ds(k * 16, 16)
                b0[i, s] = b0[i, s] + b1[i, s]
            return c
        lax.fori_loop(0, CC, _row, 0)
        pltpu.sync_copy(b0, out_hbm.at[pl.ds(tbase + j * CC, CC)])


def _run_router(flat, Wr):
    return pl.pallas_call(
        _router_body,
        grid=(T // TOK_BLK,),
        in_specs=[
            pl.BlockSpec((TOK_BLK, HIDDEN), lambda t: (t, 0)),
            pl.BlockSpec((E, HIDDEN), lambda t: (0, 0)),
        ],
        out_specs=[
            pl.BlockSpec((2, TOK_BLK), lambda t: (0, t)),
            pl.BlockSpec((2, TOK_BLK), lambda t: (0, t)),
        ],
        out_shape=[
            jax.ShapeDtypeStruct((2, T), jnp.int32),
            jax.ShapeDtypeStruct((2, T), jnp.float32),
        ],
    )(flat, Wr)


def _run_dispatch(eidx, ew):
    mesh1 = plsc.VectorSubcoreMesh(core_axis_name="c", subcore_axis_name="s",
                                   num_cores=1, num_subcores=NTILE)
    dispatch = functools.partial(
        pl.kernel,
        out_type=[
            jax.ShapeDtypeStruct((PROWS, PCOLS), jnp.int32),
            jax.ShapeDtypeStruct((PROWS, PCOLS), jnp.int32),
            jax.ShapeDtypeStruct((A,), jnp.int32),
            jax.ShapeDtypeStruct((NBP,), jnp.int32),
        ],
        mesh=mesh1,
        scratch_types=[
            pltpu.VMEM((CH,), jnp.int32),
            pltpu.VMEM((CH,), jnp.float32),
            pltpu.VMEM((CH,), jnp.int32),
            pltpu.VMEM((PROWS, PCOLS), jnp.int32),
            pltpu.VMEM((PROWS, PCOLS), jnp.int32),
            pltpu.VMEM((16,), jnp.int32),
            pltpu.VMEM((NTILE * 16,), jnp.int32),
            pltpu.VMEM((16,), jnp.int32),
            pltpu.VMEM((NBP,), jnp.int32),
            pltpu.VMEM((PROWS,), jnp.int32),
            pltpu.VMEM_SHARED((NTILE * 16,), jnp.int32),
            pltpu.VMEM_SHARED((PROWS, PCOLS), jnp.int32),
            pltpu.VMEM_SHARED((PROWS, PCOLS), jnp.int32),
        ],
        compiler_params=pltpu.CompilerParams(needs_layout_passes=False),
    )(_dispatch_body)
    gidx2, gw2i, pos, bexp = dispatch(eidx.reshape(A), ew.reshape(A))
    gidx = gidx2.reshape(P)
    gw2 = lax.bitcast_convert_type(gw2i, jnp.float32)
    return gidx, gw2, pos, bexp


def _run_gather(flat, gidx):
    mesh2 = plsc.VectorSubcoreMesh(core_axis_name="c", subcore_axis_name="s",
                                   num_cores=2, num_subcores=NTILE)
    gather = functools.partial(
        pl.kernel,
        out_type=jax.ShapeDtypeStruct((P, HIDDEN), jnp.float32),
        mesh=mesh2,
        scratch_types=[
            pltpu.VMEM((GSL,), jnp.int32),
            pltpu.VMEM((GCH, HIDDEN), jnp.float32),
            pltpu.VMEM((GCH, HIDDEN), jnp.float32),
            pltpu.SemaphoreType.DMA,
            pltpu.SemaphoreType.DMA,
        ],
        compiler_params=pltpu.CompilerParams(needs_layout_passes=False),
    )(_gather_body)
    return gather(flat, gidx)


def _run_ffn(xg, W1, b1, W2, b2, gw2, bexp):
    h_all = pl.pallas_call(
        _ffn1_body,
        grid_spec=pltpu.PrefetchScalarGridSpec(
            num_scalar_prefetch=1,
            grid=(NB,),
            in_specs=[
                pl.BlockSpec((BLK, HIDDEN), lambda b, be: (b, 0)),
                pl.BlockSpec((1, FFN, HIDDEN), lambda b, be: (be[b], 0, 0)),
                pl.BlockSpec((1, 1, FFN), lambda b, be: (be[b], 0, 0)),
            ],
            out_specs=pl.BlockSpec((BLK, FFN), lambda b, be: (b, 0)),
        ),
        out_shape=jax.ShapeDtypeStruct((P, FFN), jnp.float32),
    )(bexp, xg, W1, b1.reshape(E, 1, FFN))

    og = pl.pallas_call(
        _ffn2_body,
        grid_spec=pltpu.PrefetchScalarGridSpec(
            num_scalar_prefetch=1,
            grid=(NB,),
            in_specs=[
                pl.BlockSpec((BLK, FFN), lambda b, be: (b, 0)),
                pl.BlockSpec((1, HIDDEN, FFN), lambda b, be: (be[b], 0, 0)),
                pl.BlockSpec((1, 1, HIDDEN), lambda b, be: (be[b], 0, 0)),
                pl.BlockSpec((1, 1, BLK), lambda b, be: (b, 0, 0)),
            ],
            out_specs=pl.BlockSpec((BLK, HIDDEN), lambda b, be: (b, 0)),
        ),
        out_shape=jax.ShapeDtypeStruct((P, HIDDEN), jnp.float32),
    )(bexp, h_all, W2, b2.reshape(E, 1, HIDDEN), gw2.reshape(NB, 1, BLK))
    return og


def _run_combine(og, pos):
    mesh2 = plsc.VectorSubcoreMesh(core_axis_name="c", subcore_axis_name="s",
                                   num_cores=2, num_subcores=NTILE)
    combine = functools.partial(
        pl.kernel,
        out_type=jax.ShapeDtypeStruct((T, HIDDEN), jnp.float32),
        mesh=mesh2,
        scratch_types=[
            pltpu.VMEM((TPT,), jnp.int32),
            pltpu.VMEM((TPT,), jnp.int32),
            pltpu.VMEM((CC, HIDDEN), jnp.float32),
            pltpu.VMEM((CC, HIDDEN), jnp.float32),
            pltpu.VMEM((CC, HIDDEN), jnp.float32),
            pltpu.VMEM((CC, HIDDEN), jnp.float32),
            pltpu.SemaphoreType.DMA,
            pltpu.SemaphoreType.DMA,
        ],
        compiler_params=pltpu.CompilerParams(needs_layout_passes=False),
    )(_combine_body)
    return combine(og, pos)


def kernel(x, Wr, W1, b1, W2, b2):
    batch, seq, hidden = x.shape
    flat = x.reshape(T, hidden)
    eidx, ew = _run_router(flat, Wr)
    gidx, gw2, pos, bexp = _run_dispatch(eidx, ew)
    xg = _run_gather(flat, gidx)
    og = _run_ffn(xg, W1, b1, W2, b2, gw2, bexp)
    out = _run_combine(og, pos)
    return out.reshape(batch, seq, hidden)


# speedup vs baseline: 1.7546x; 1.0321x over previous
import functools

import jax
import jax.numpy as jnp
from jax import lax
from jax.experimental import pallas as pl
from jax.experimental.pallas import tpu as pltpu
from jax.experimental.pallas import tpu_sc as plsc

HIDDEN = 1024
FFN = 4096
E = 8
T = 4096
A = 2 * T
BLK = 256
P = A + E * BLK
NB = P // BLK
NBP = 48
TOK_BLK = 512

NTILE = 16
CH = A // NTILE
NG = CH // 16
PCOLS = 128
PROWS = P // PCOLS

GT = 32
GSL = P // GT
GCH = 80
GD = HIDDEN // 2
TPT = T // GT
CC = 16


def _router_body(x_ref, wr_ref, eidx_ref, ew_ref):
    xb = x_ref[...]
    logits = lax.dot_general(xb, wr_ref[...], (((1,), (1,)), ((), ())),
                             preferred_element_type=jnp.float32)
    m = jnp.max(logits, axis=1, keepdims=True)
    ex = jnp.exp(logits - m)
    probs = ex / jnp.sum(ex, axis=1, keepdims=True)

    iota = lax.broadcasted_iota(jnp.int32, probs.shape, 1)
    m1 = jnp.max(probs, axis=1, keepdims=True)
    idx1 = jnp.min(jnp.where(probs == m1, iota, E), axis=1, keepdims=True)
    p2 = jnp.where(iota == idx1, -jnp.inf, probs)
    m2 = jnp.max(p2, axis=1, keepdims=True)
    idx2 = jnp.min(jnp.where(p2 == m2, iota, E), axis=1, keepdims=True)
    eidx_ref[...] = jnp.concatenate([idx1.T, idx2.T], axis=0)
    ew_ref[...] = jnp.concatenate([m1.T, m2.T], axis=0)


def _dispatch_body(eidx_hbm, ew_hbm, gidx_hbm, gw_hbm, pos_hbm, bexp_hbm,
                   ids_v, ws_v, pos_v, gidx_v, gw_v, vec_v, all_v, cur_v,
                   bexp_v, rowi_v, sh_cnt, sh_gidx, sh_gw):
    wid = lax.axis_index("s")
    base = wid * CH
    lane = lax.iota(jnp.int32, 16)
    z16i = jnp.zeros((16,), jnp.int32)

    pltpu.sync_copy(eidx_hbm.at[pl.ds(base, CH)], ids_v)
    pltpu.sync_copy(ew_hbm.at[pl.ds(base, CH)], ws_v)

    def _zrow(i, c):
        for k in range(PCOLS // 16):
            gidx_v[i, pl.ds(k * 16, 16)] = z16i
            gw_v[i, pl.ds(k * 16, 16)] = z16i
        return c
    lax.fori_loop(0, PROWS, _zrow, 0)

    def _riota(j, c):
        rowi_v[pl.ds(j * 16, 16)] = j * 16 + lane
        return c
    lax.fori_loop(0, PROWS // 16, _riota, 0)

    def _hist(g, cnt):
        ids16 = ids_v[pl.ds(g * 16, 16)]
        for e in range(E):
            c = jnp.sum((ids16 == e).astype(jnp.int32))
            cnt = cnt + jnp.where(lane == e, c, 0)
        return cnt
    cnt = lax.fori_loop(0, NG, _hist, z16i)
    vec_v[...] = cnt
    pltpu.sync_copy(vec_v, sh_cnt.at[pl.ds(wid * 16, 16)])

    @pl.when(wid == 0)
    def _():
        pltpu.sync_copy(gidx_v, sh_gidx)
        pltpu.sync_copy(gw_v, sh_gw)

    plsc.subcore_barrier()

    pltpu.sync_copy(sh_cnt, all_v)
    tot = z16i
    pre = z16i
    for w in range(NTILE):
        row = all_v[pl.ds(w * 16, 16)]
        tot = tot + row
        pre = pre + jnp.where(w < wid, row, z16i)
    padded = ((tot + (BLK - 1)) >> 8) << 8
    inc = plsc.cumsum(padded)
    off = inc - padded
    cur_v[...] = off + pre

    @pl.when(wid == 0)
    def _():
        binc = plsc.cumsum(padded >> 8)
        nbu = jnp.sum(jnp.where(lane == E - 1, binc, 0))
        for c in range(NBP // 16):
            bv = lane + c * 16
            acc = z16i
            for e in range(E):
                s_e = jnp.sum(jnp.where(lane == e, binc, 0))
                acc = acc + (bv >= s_e).astype(jnp.int32)
            ch = jnp.minimum(acc, E - 1)
            if c * 16 <= NB < (c + 1) * 16:
                ch = jnp.where(lane == NB - c * 16, nbu, ch)
            bexp_v[pl.ds(c * 16, 16)] = ch
        pltpu.sync_copy(bexp_v, bexp_hbm)

    def _assign(g, c):
        ids16 = ids_v[pl.ds(g * 16, 16)]
        ws16 = ws_v[pl.ds(g * 16, 16)]
        tok16 = (base + g * 16 + lane) & (T - 1)
        curv = plsc.load_gather(cur_v, [ids16])
        rank = z16i
        upd = z16i
        for e in range(E):
            oh = ids16 == e
            ohi = oh.astype(jnp.int32)
            cs = plsc.cumsum(ohi)
            rank = rank + jnp.where(oh, cs - 1, z16i)
            upd = upd + jnp.where(lane == e, jnp.sum(ohi), 0)
        dest = curv + rank
        cur_v[...] = cur_v[...] + upd
        plsc.store_scatter(gidx_v, [dest >> 7, dest & (PCOLS - 1)], tok16)
        plsc.store_scatter(gw_v, [dest >> 7, dest & (PCOLS - 1)],
                           plsc.bitcast(ws16, jnp.int32))
        pos_v[pl.ds(g * 16, 16)] = dest
        return c
    lax.fori_loop(0, NG, _assign, 0)

    pltpu.sync_copy(pos_v, pos_hbm.at[pl.ds(base, CH)])

    plsc.subcore_barrier()
    pltpu.sync_copy(gidx_v, sh_gidx.at[rowi_v], add=True)
    pltpu.sync_copy(gw_v, sh_gw.at[rowi_v], add=True)
    plsc.subcore_barrier()

    @pl.when(wid < PROWS // 8)
    def _():
        pltpu.sync_copy(sh_gidx.at[pl.ds(wid * 8, 8)],
                        gidx_hbm.at[pl.ds(wid * 8, 8)])
        pltpu.sync_copy(sh_gw.at[pl.ds(wid * 8, 8)],
                        gw_hbm.at[pl.ds(wid * 8, 8)])


def _gather_body(flat_hbm, gidx_hbm, xg_hbm, idx_v, rows0_v, rows1_v, sem0,
                 sem1):
    wid = lax.axis_index("s") * 2 + lax.axis_index("c")
    base = wid * GSL
    nch = GSL // GCH
    bufs = (rows0_v, rows1_v)
    sems = (sem0, sem1)
    pltpu.sync_copy(gidx_hbm.at[pl.ds(base, GSL)], idx_v)
    cps = [None, None]
    cps[0] = pltpu.async_copy(flat_hbm.at[idx_v.at[pl.ds(0, GCH)]],
                              bufs[0], sems[0])
    for j in range(nch):
        if j + 1 < nch:
            cps[(j + 1) % 2] = pltpu.async_copy(
                flat_hbm.at[idx_v.at[pl.ds((j + 1) * GCH, GCH)]],
                bufs[(j + 1) % 2], sems[(j + 1) % 2])
        cps[j % 2].wait()
        pltpu.sync_copy(bufs[j % 2], xg_hbm.at[pl.ds(base + j * GCH, GCH)])


def _ffn1_body(bexp_ref, xg_ref, w1_ref, b1_ref, h_ref):
    @pl.when(pl.program_id(0) < bexp_ref[NB])
    def _():
        xi = xg_ref[...]
        xlo = lax.bitcast_convert_type(xi << 16, jnp.float32)
        xhi = lax.bitcast_convert_type(xi & jnp.int32(-65536), jnp.float32)
        w1 = w1_ref[0]
        h = lax.dot_general(xlo, w1[:, :GD], (((1,), (1,)), ((), ())),
                            preferred_element_type=jnp.float32)
        h = h + lax.dot_general(xhi, w1[:, GD:], (((1,), (1,)), ((), ())),
                                preferred_element_type=jnp.float32)
        h = h + b1_ref[0]
        h_ref[...] = h * jax.nn.sigmoid(h)


def _ffn2_body(bexp_ref, h_ref, w2_ref, b2_ref, gw_ref, og_ref):
    @pl.when(pl.program_id(0) < bexp_ref[NB])
    def _():
        o = lax.dot_general(h_ref[...], w2_ref[0], (((1,), (1,)), ((), ())),
                            preferred_element_type=jnp.float32)
        o = o + b2_ref[0]
        og_ref[...] = o * gw_ref[0, 0][:, None]


def _combine_body(og_hbm, pos_hbm, out_hbm, idx0_v, idx1_v, bufa0_v, bufa1_v,
                  bufb0_v, bufb1_v, sem0, sem1):
    wid = lax.axis_index("s") * 2 + lax.axis_index("c")
    tbase = wid * TPT
    nch = TPT // CC
    bufs = ((bufa0_v, bufa1_v), (bufb0_v, bufb1_v))
    sems = (sem0, sem1)
    pltpu.sync_copy(pos_hbm.at[pl.ds(tbase, TPT)], idx0_v)
    pltpu.sync_copy(pos_hbm.at[pl.ds(T + tbase, TPT)], idx1_v)

    def _start(j):
        b0, b1 = bufs[j % 2]
        s = sems[j % 2]
        c0 = pltpu.async_copy(og_hbm.at[idx0_v.at[pl.ds(j * CC, CC)]], b0, s)
        c1 = pltpu.async_copy(og_hbm.at[idx1_v.at[pl.ds(j * CC, CC)]], b1, s)
        return (c0, c1)

    cps = [None, None]
    cps[0] = _start(0)
    for j in range(nch):
        if j + 1 < nch:
            cps[(j + 1) % 2] = _start(j + 1)
        cps[j % 2][0].wait()
        cps[j % 2][1].wait()
        b0, b1 = bufs[j % 2]

        def _row(i, c):
            for k in range(HIDDEN // 16):
                s = pl.ds(k * 16, 16)
                b0[i, s] = b0[i, s] + b1[i, s]
            return c
        lax.fori_loop(0, CC, _row, 0)
        pltpu.sync_copy(b0, out_hbm.at[pl.ds(tbase + j * CC, CC)])


def _run_router(flat, Wr):
    return pl.pallas_call(
        _router_body,
        grid=(T // TOK_BLK,),
        in_specs=[
            pl.BlockSpec((TOK_BLK, HIDDEN), lambda t: (t, 0)),
            pl.BlockSpec((E, HIDDEN), lambda t: (0, 0)),
        ],
        out_specs=[
            pl.BlockSpec((2, TOK_BLK), lambda t: (0, t)),
            pl.BlockSpec((2, TOK_BLK), lambda t: (0, t)),
        ],
        out_shape=[
            jax.ShapeDtypeStruct((2, T), jnp.int32),
            jax.ShapeDtypeStruct((2, T), jnp.float32),
        ],
    )(flat, Wr)


def _run_dispatch(eidx, ew):
    mesh1 = plsc.VectorSubcoreMesh(core_axis_name="c", subcore_axis_name="s",
                                   num_cores=1, num_subcores=NTILE)
    dispatch = functools.partial(
        pl.kernel,
        out_type=[
            jax.ShapeDtypeStruct((PROWS, PCOLS), jnp.int32),
            jax.ShapeDtypeStruct((PROWS, PCOLS), jnp.int32),
            jax.ShapeDtypeStruct((A,), jnp.int32),
            jax.ShapeDtypeStruct((NBP,), jnp.int32),
        ],
        mesh=mesh1,
        scratch_types=[
            pltpu.VMEM((CH,), jnp.int32),
            pltpu.VMEM((CH,), jnp.float32),
            pltpu.VMEM((CH,), jnp.int32),
            pltpu.VMEM((PROWS, PCOLS), jnp.int32),
            pltpu.VMEM((PROWS, PCOLS), jnp.int32),
            pltpu.VMEM((16,), jnp.int32),
            pltpu.VMEM((NTILE * 16,), jnp.int32),
            pltpu.VMEM((16,), jnp.int32),
            pltpu.VMEM((NBP,), jnp.int32),
            pltpu.VMEM((PROWS,), jnp.int32),
            pltpu.VMEM_SHARED((NTILE * 16,), jnp.int32),
            pltpu.VMEM_SHARED((PROWS, PCOLS), jnp.int32),
            pltpu.VMEM_SHARED((PROWS, PCOLS), jnp.int32),
        ],
        compiler_params=pltpu.CompilerParams(needs_layout_passes=False),
    )(_dispatch_body)
    gidx2, gw2i, pos, bexp = dispatch(eidx.reshape(A), ew.reshape(A))
    gidx = gidx2.reshape(P)
    gw2 = lax.bitcast_convert_type(gw2i, jnp.float32)
    return gidx, gw2, pos, bexp


def _run_gather(flat, gidx):
    mesh2 = plsc.VectorSubcoreMesh(core_axis_name="c", subcore_axis_name="s",
                                   num_cores=2, num_subcores=NTILE)
    gather = functools.partial(
        pl.kernel,
        out_type=jax.ShapeDtypeStruct((P, GD), jnp.int32),
        mesh=mesh2,
        scratch_types=[
            pltpu.VMEM((GSL,), jnp.int32),
            pltpu.VMEM((GCH, GD), jnp.int32),
            pltpu.VMEM((GCH, GD), jnp.int32),
            pltpu.SemaphoreType.DMA,
            pltpu.SemaphoreType.DMA,
        ],
        compiler_params=pltpu.CompilerParams(needs_layout_passes=False),
    )(_gather_body)
    return gather(flat, gidx)


def _run_ffn(xg, W1, b1, W2, b2, gw2, bexp):
    h_all = pl.pallas_call(
        _ffn1_body,
        grid_spec=pltpu.PrefetchScalarGridSpec(
            num_scalar_prefetch=1,
            grid=(NB,),
            in_specs=[
                pl.BlockSpec((BLK, GD), lambda b, be: (b, 0)),
                pl.BlockSpec((1, FFN, HIDDEN), lambda b, be: (be[b], 0, 0)),
                pl.BlockSpec((1, 1, FFN), lambda b, be: (be[b], 0, 0)),
            ],
            out_specs=pl.BlockSpec((BLK, FFN), lambda b, be: (b, 0)),
        ),
        out_shape=jax.ShapeDtypeStruct((P, FFN), jnp.float32),
    )(bexp, xg, W1, b1.reshape(E, 1, FFN))

    og = pl.pallas_call(
        _ffn2_body,
        grid_spec=pltpu.PrefetchScalarGridSpec(
            num_scalar_prefetch=1,
            grid=(NB,),
            in_specs=[
                pl.BlockSpec((BLK, FFN), lambda b, be: (b, 0)),
                pl.BlockSpec((1, HIDDEN, FFN), lambda b, be: (be[b], 0, 0)),
                pl.BlockSpec((1, 1, HIDDEN), lambda b, be: (be[b], 0, 0)),
                pl.BlockSpec((1, 1, BLK), lambda b, be: (b, 0, 0)),
            ],
            out_specs=pl.BlockSpec((BLK, HIDDEN), lambda b, be: (b, 0)),
        ),
        out_shape=jax.ShapeDtypeStruct((P, HIDDEN), jnp.float32),
    )(bexp, h_all, W2, b2.reshape(E, 1, HIDDEN), gw2.reshape(NB, 1, BLK))
    return og


def _run_combine(og, pos):
    mesh2 = plsc.VectorSubcoreMesh(core_axis_name="c", subcore_axis_name="s",
                                   num_cores=2, num_subcores=NTILE)
    combine = functools.partial(
        pl.kernel,
        out_type=jax.ShapeDtypeStruct((T, HIDDEN), jnp.float32),
        mesh=mesh2,
        scratch_types=[
            pltpu.VMEM((TPT,), jnp.int32),
            pltpu.VMEM((TPT,), jnp.int32),
            pltpu.VMEM((CC, HIDDEN), jnp.float32),
            pltpu.VMEM((CC, HIDDEN), jnp.float32),
            pltpu.VMEM((CC, HIDDEN), jnp.float32),
            pltpu.VMEM((CC, HIDDEN), jnp.float32),
            pltpu.SemaphoreType.DMA,
            pltpu.SemaphoreType.DMA,
        ],
        compiler_params=pltpu.CompilerParams(needs_layout_passes=False),
    )(_combine_body)
    return combine(og, pos)


def kernel(x, Wr, W1, b1, W2, b2):
    batch, seq, hidden = x.shape
    flat = x.reshape(T, hidden)
    eidx, ew = _run_router(flat, Wr)
    gidx, gw2, pos, bexp = _run_dispatch(eidx, ew)
    flatb = flat.astype(jnp.bfloat16)
    flat_pack = lax.bitcast_convert_type(
        jnp.stack([flatb[:, :GD], flatb[:, GD:]], axis=-1), jnp.int32)
    xg = _run_gather(flat_pack, gidx)
    og = _run_ffn(xg, W1, b1, W2, b2, gw2, bexp)
    out = _run_combine(og, pos)
    return out.reshape(batch, seq, hidden)


# speedup vs baseline: 1.7680x; 1.0076x over previous
import functools

import jax
import jax.numpy as jnp
from jax import lax
from jax.experimental import pallas as pl
from jax.experimental.pallas import tpu as pltpu
from jax.experimental.pallas import tpu_sc as plsc

HIDDEN = 1024
FFN = 4096
E = 8
T = 4096
A = 2 * T
BLK = 256
P = A + E * BLK
NB = P // BLK
NBP = 48
TOK_BLK = 512

NTILE = 16
CH = A // NTILE
NG = CH // 16
PCOLS = 128
PROWS = P // PCOLS

GT = 32
GSL = P // GT
GCH = 40
GD = HIDDEN // 2
TPT = T // GT
CC = 16


def _router_body(x_ref, wr_ref, eidx_ref, ew_ref):
    xb = x_ref[...]
    logits = lax.dot_general(xb, wr_ref[...], (((1,), (1,)), ((), ())),
                             preferred_element_type=jnp.float32)
    m = jnp.max(logits, axis=1, keepdims=True)
    ex = jnp.exp(logits - m)
    probs = ex / jnp.sum(ex, axis=1, keepdims=True)

    iota = lax.broadcasted_iota(jnp.int32, probs.shape, 1)
    m1 = jnp.max(probs, axis=1, keepdims=True)
    idx1 = jnp.min(jnp.where(probs == m1, iota, E), axis=1, keepdims=True)
    p2 = jnp.where(iota == idx1, -jnp.inf, probs)
    m2 = jnp.max(p2, axis=1, keepdims=True)
    idx2 = jnp.min(jnp.where(p2 == m2, iota, E), axis=1, keepdims=True)
    eidx_ref[...] = jnp.concatenate([idx1.T, idx2.T], axis=0)
    ew_ref[...] = jnp.concatenate([m1.T, m2.T], axis=0)


def _dispatch_body(eidx_hbm, ew_hbm, gidx_hbm, gw_hbm, pos_hbm, bexp_hbm,
                   ids_v, ws_v, pos_v, gidx_v, gw_v, vec_v, all_v, cur_v,
                   bexp_v, rowi_v, sh_cnt, sh_gidx, sh_gw):
    wid = lax.axis_index("s")
    base = wid * CH
    lane = lax.iota(jnp.int32, 16)
    z16i = jnp.zeros((16,), jnp.int32)

    pltpu.sync_copy(eidx_hbm.at[pl.ds(base, CH)], ids_v)
    pltpu.sync_copy(ew_hbm.at[pl.ds(base, CH)], ws_v)

    def _zrow(i, c):
        for k in range(PCOLS // 16):
            gidx_v[i, pl.ds(k * 16, 16)] = z16i
            gw_v[i, pl.ds(k * 16, 16)] = z16i
        return c
    lax.fori_loop(0, PROWS, _zrow, 0)

    def _riota(j, c):
        rowi_v[pl.ds(j * 16, 16)] = j * 16 + lane
        return c
    lax.fori_loop(0, PROWS // 16, _riota, 0)

    def _hist(g, cnt):
        ids16 = ids_v[pl.ds(g * 16, 16)]
        for e in range(E):
            c = jnp.sum((ids16 == e).astype(jnp.int32))
            cnt = cnt + jnp.where(lane == e, c, 0)
        return cnt
    cnt = lax.fori_loop(0, NG, _hist, z16i)
    vec_v[...] = cnt
    pltpu.sync_copy(vec_v, sh_cnt.at[pl.ds(wid * 16, 16)])

    @pl.when(wid == 0)
    def _():
        pltpu.sync_copy(gidx_v, sh_gidx)
        pltpu.sync_copy(gw_v, sh_gw)

    plsc.subcore_barrier()

    pltpu.sync_copy(sh_cnt, all_v)
    tot = z16i
    pre = z16i
    for w in range(NTILE):
        row = all_v[pl.ds(w * 16, 16)]
        tot = tot + row
        pre = pre + jnp.where(w < wid, row, z16i)
    padded = ((tot + (BLK - 1)) >> 8) << 8
    inc = plsc.cumsum(padded)
    off = inc - padded
    cur_v[...] = off + pre

    @pl.when(wid == 0)
    def _():
        binc = plsc.cumsum(padded >> 8)
        nbu = jnp.sum(jnp.where(lane == E - 1, binc, 0))
        for c in range(NBP // 16):
            bv = lane + c * 16
            acc = z16i
            for e in range(E):
                s_e = jnp.sum(jnp.where(lane == e, binc, 0))
                acc = acc + (bv >= s_e).astype(jnp.int32)
            ch = jnp.minimum(acc, E - 1)
            if c * 16 <= NB < (c + 1) * 16:
                ch = jnp.where(lane == NB - c * 16, nbu, ch)
            bexp_v[pl.ds(c * 16, 16)] = ch
        pltpu.sync_copy(bexp_v, bexp_hbm)

    def _assign(g, c):
        ids16 = ids_v[pl.ds(g * 16, 16)]
        ws16 = ws_v[pl.ds(g * 16, 16)]
        tok16 = (base + g * 16 + lane) & (T - 1)
        curv = plsc.load_gather(cur_v, [ids16])
        rank = z16i
        upd = z16i
        for e in range(E):
            oh = ids16 == e
            ohi = oh.astype(jnp.int32)
            cs = plsc.cumsum(ohi)
            rank = rank + jnp.where(oh, cs - 1, z16i)
            upd = upd + jnp.where(lane == e, jnp.sum(ohi), 0)
        dest = curv + rank
        cur_v[...] = cur_v[...] + upd
        plsc.store_scatter(gidx_v, [dest >> 7, dest & (PCOLS - 1)], tok16)
        plsc.store_scatter(gw_v, [dest >> 7, dest & (PCOLS - 1)],
                           plsc.bitcast(ws16, jnp.int32))
        pos_v[pl.ds(g * 16, 16)] = dest
        return c
    lax.fori_loop(0, NG, _assign, 0)

    pltpu.sync_copy(pos_v, pos_hbm.at[pl.ds(base, CH)])

    plsc.subcore_barrier()
    pltpu.sync_copy(gidx_v, sh_gidx.at[rowi_v], add=True)
    pltpu.sync_copy(gw_v, sh_gw.at[rowi_v], add=True)
    plsc.subcore_barrier()

    @pl.when(wid < PROWS // 8)
    def _():
        pltpu.sync_copy(sh_gidx.at[pl.ds(wid * 8, 8)],
                        gidx_hbm.at[pl.ds(wid * 8, 8)])
        pltpu.sync_copy(sh_gw.at[pl.ds(wid * 8, 8)],
                        gw_hbm.at[pl.ds(wid * 8, 8)])


def _gather_body(flat_hbm, gidx_hbm, xg_hbm, idx_v, rows0_v, rows1_v,
                 rows2_v, sem0, sem1, sem2, ssem0, ssem1, ssem2):
    wid = lax.axis_index("s") * 2 + lax.axis_index("c")
    base = wid * GSL
    nch = GSL // GCH
    bufs = (rows0_v, rows1_v, rows2_v)
    gsems = (sem0, sem1, sem2)
    ssems = (ssem0, ssem1, ssem2)
    pltpu.sync_copy(gidx_hbm.at[pl.ds(base, GSL)], idx_v)

    def _g(j):
        return pltpu.async_copy(
            flat_hbm.at[idx_v.at[pl.ds(j * GCH, GCH)]],
            bufs[j % 3], gsems[j % 3])

    gcp = [None, None, None]
    scp = [None, None, None]
    gcp[0] = _g(0)
    if nch > 1:
        gcp[1] = _g(1)
    for j in range(nch):
        b = j % 3
        nxt = j + 2
        if nxt < nch:
            nb = nxt % 3
            if scp[nb] is not None:
                scp[nb].wait()
                scp[nb] = None
            gcp[nb] = _g(nxt)
        gcp[b].wait()
        scp[b] = pltpu.async_copy(
            bufs[b], xg_hbm.at[pl.ds(base + j * GCH, GCH)], ssems[b])
    for cp in scp:
        if cp is not None:
            cp.wait()


def _ffn1_body(bexp_ref, xg_ref, w1_ref, b1_ref, h_ref):
    @pl.when(pl.program_id(0) < bexp_ref[NB])
    def _():
        xi = xg_ref[...]
        xlo = lax.bitcast_convert_type(xi << 16, jnp.float32)
        xhi = lax.bitcast_convert_type(xi & jnp.int32(-65536), jnp.float32)
        w1 = w1_ref[0]
        h = lax.dot_general(xlo, w1[:, :GD], (((1,), (1,)), ((), ())),
                            preferred_element_type=jnp.float32)
        h = h + lax.dot_general(xhi, w1[:, GD:], (((1,), (1,)), ((), ())),
                                preferred_element_type=jnp.float32)
        h = h + b1_ref[0]
        h_ref[...] = h * jax.nn.sigmoid(h)


def _ffn2_body(bexp_ref, h_ref, w2_ref, b2_ref, gw_ref, og_ref):
    @pl.when(pl.program_id(0) < bexp_ref[NB])
    def _():
        o = lax.dot_general(h_ref[...], w2_ref[0], (((1,), (1,)), ((), ())),
                            preferred_element_type=jnp.float32)
        o = o + b2_ref[0]
        og_ref[...] = o * gw_ref[0, 0][:, None]


def _combine_body(og_hbm, pos_hbm, out_hbm, idx0_v, idx1_v, bufa0_v, bufa1_v,
                  bufb0_v, bufb1_v, sem0, sem1):
    wid = lax.axis_index("s") * 2 + lax.axis_index("c")
    tbase = wid * TPT
    nch = TPT // CC
    bufs = ((bufa0_v, bufa1_v), (bufb0_v, bufb1_v))
    sems = (sem0, sem1)
    pltpu.sync_copy(pos_hbm.at[pl.ds(tbase, TPT)], idx0_v)
    pltpu.sync_copy(pos_hbm.at[pl.ds(T + tbase, TPT)], idx1_v)

    def _start(j):
        b0, b1 = bufs[j % 2]
        s = sems[j % 2]
        c0 = pltpu.async_copy(og_hbm.at[idx0_v.at[pl.ds(j * CC, CC)]], b0, s)
        c1 = pltpu.async_copy(og_hbm.at[idx1_v.at[pl.ds(j * CC, CC)]], b1, s)
        return (c0, c1)

    cps = [None, None]
    cps[0] = _start(0)
    for j in range(nch):
        if j + 1 < nch:
            cps[(j + 1) % 2] = _start(j + 1)
        cps[j % 2][0].wait()
        cps[j % 2][1].wait()
        b0, b1 = bufs[j % 2]

        def _row(i, c):
            for k in range(HIDDEN // 16):
                s = pl.ds(k * 16, 16)
                b0[i, s] = b0[i, s] + b1[i, s]
            return c
        lax.fori_loop(0, CC, _row, 0)
        pltpu.sync_copy(b0, out_hbm.at[pl.ds(tbase + j * CC, CC)])


def _run_router(flat, Wr):
    return pl.pallas_call(
        _router_body,
        grid=(T // TOK_BLK,),
        in_specs=[
            pl.BlockSpec((TOK_BLK, HIDDEN), lambda t: (t, 0)),
            pl.BlockSpec((E, HIDDEN), lambda t: (0, 0)),
        ],
        out_specs=[
            pl.BlockSpec((2, TOK_BLK), lambda t: (0, t)),
            pl.BlockSpec((2, TOK_BLK), lambda t: (0, t)),
        ],
        out_shape=[
            jax.ShapeDtypeStruct((2, T), jnp.int32),
            jax.ShapeDtypeStruct((2, T), jnp.float32),
        ],
    )(flat, Wr)


def _run_dispatch(eidx, ew):
    mesh1 = plsc.VectorSubcoreMesh(core_axis_name="c", subcore_axis_name="s",
                                   num_cores=1, num_subcores=NTILE)
    dispatch = functools.partial(
        pl.kernel,
        out_type=[
            jax.ShapeDtypeStruct((PROWS, PCOLS), jnp.int32),
            jax.ShapeDtypeStruct((PROWS, PCOLS), jnp.int32),
            jax.ShapeDtypeStruct((A,), jnp.int32),
            jax.ShapeDtypeStruct((NBP,), jnp.int32),
        ],
        mesh=mesh1,
        scratch_types=[
            pltpu.VMEM((CH,), jnp.int32),
            pltpu.VMEM((CH,), jnp.float32),
            pltpu.VMEM((CH,), jnp.int32),
            pltpu.VMEM((PROWS, PCOLS), jnp.int32),
            pltpu.VMEM((PROWS, PCOLS), jnp.int32),
            pltpu.VMEM((16,), jnp.int32),
            pltpu.VMEM((NTILE * 16,), jnp.int32),
            pltpu.VMEM((16,), jnp.int32),
            pltpu.VMEM((NBP,), jnp.int32),
            pltpu.VMEM((PROWS,), jnp.int32),
            pltpu.VMEM_SHARED((NTILE * 16,), jnp.int32),
            pltpu.VMEM_SHARED((PROWS, PCOLS), jnp.int32),
            pltpu.VMEM_SHARED((PROWS, PCOLS), jnp.int32),
        ],
        compiler_params=pltpu.CompilerParams(needs_layout_passes=False),
    )(_dispatch_body)
    gidx2, gw2i, pos, bexp = dispatch(eidx.reshape(A), ew.reshape(A))
    gidx = gidx2.reshape(P)
    gw2 = lax.bitcast_convert_type(gw2i, jnp.float32)
    return gidx, gw2, pos, bexp


def _run_gather(flat, gidx):
    mesh2 = plsc.VectorSubcoreMesh(core_axis_name="c", subcore_axis_name="s",
                                   num_cores=2, num_subcores=NTILE)
    gather = functools.partial(
        pl.kernel,
        out_type=jax.ShapeDtypeStruct((P, GD), jnp.int32),
        mesh=mesh2,
        scratch_types=[
            pltpu.VMEM((GSL,), jnp.int32),
            pltpu.VMEM((GCH, GD), jnp.int32),
            pltpu.VMEM((GCH, GD), jnp.int32),
            pltpu.VMEM((GCH, GD), jnp.int32),
            pltpu.SemaphoreType.DMA,
            pltpu.SemaphoreType.DMA,
            pltpu.SemaphoreType.DMA,
            pltpu.SemaphoreType.DMA,
            pltpu.SemaphoreType.DMA,
            pltpu.SemaphoreType.DMA,
        ],
        compiler_params=pltpu.CompilerParams(needs_layout_passes=False),
    )(_gather_body)
    return gather(flat, gidx)


def _run_ffn(xg, W1, b1, W2, b2, gw2, bexp):
    h_all = pl.pallas_call(
        _ffn1_body,
        grid_spec=pltpu.PrefetchScalarGridSpec(
            num_scalar_prefetch=1,
            grid=(NB,),
            in_specs=[
                pl.BlockSpec((BLK, GD), lambda b, be: (b, 0)),
                pl.BlockSpec((1, FFN, HIDDEN), lambda b, be: (be[b], 0, 0)),
                pl.BlockSpec((1, 1, FFN), lambda b, be: (be[b], 0, 0)),
            ],
            out_specs=pl.BlockSpec((BLK, FFN), lambda b, be: (b, 0)),
        ),
        out_shape=jax.ShapeDtypeStruct((P, FFN), jnp.float32),
    )(bexp, xg, W1, b1.reshape(E, 1, FFN))

    og = pl.pallas_call(
        _ffn2_body,
        grid_spec=pltpu.PrefetchScalarGridSpec(
            num_scalar_prefetch=1,
            grid=(NB,),
            in_specs=[
                pl.BlockSpec((BLK, FFN), lambda b, be: (b, 0)),
                pl.BlockSpec((1, HIDDEN, FFN), lambda b, be: (be[b], 0, 0)),
                pl.BlockSpec((1, 1, HIDDEN), lambda b, be: (be[b], 0, 0)),
                pl.BlockSpec((1, 1, BLK), lambda b, be: (b, 0, 0)),
            ],
            out_specs=pl.BlockSpec((BLK, HIDDEN), lambda b, be: (b, 0)),
        ),
        out_shape=jax.ShapeDtypeStruct((P, HIDDEN), jnp.float32),
    )(bexp, h_all, W2, b2.reshape(E, 1, HIDDEN), gw2.reshape(NB, 1, BLK))
    return og


def _run_combine(og, pos):
    mesh2 = plsc.VectorSubcoreMesh(core_axis_name="c", subcore_axis_name="s",
                                   num_cores=2, num_subcores=NTILE)
    combine = functools.partial(
        pl.kernel,
        out_type=jax.ShapeDtypeStruct((T, HIDDEN), jnp.float32),
        mesh=mesh2,
        scratch_types=[
            pltpu.VMEM((TPT,), jnp.int32),
            pltpu.VMEM((TPT,), jnp.int32),
            pltpu.VMEM((CC, HIDDEN), jnp.float32),
            pltpu.VMEM((CC, HIDDEN), jnp.float32),
            pltpu.VMEM((CC, HIDDEN), jnp.float32),
            pltpu.VMEM((CC, HIDDEN), jnp.float32),
            pltpu.SemaphoreType.DMA,
            pltpu.SemaphoreType.DMA,
        ],
        compiler_params=pltpu.CompilerParams(needs_layout_passes=False),
    )(_combine_body)
    return combine(og, pos)


def kernel(x, Wr, W1, b1, W2, b2):
    batch, seq, hidden = x.shape
    flat = x.reshape(T, hidden)
    eidx, ew = _run_router(flat, Wr)
    gidx, gw2, pos, bexp = _run_dispatch(eidx, ew)
    flatb = flat.astype(jnp.bfloat16)
    flat_pack = lax.bitcast_convert_type(
        jnp.stack([flatb[:, :GD], flatb[:, GD:]], axis=-1), jnp.int32)
    xg = _run_gather(flat_pack, gidx)
    og = _run_ffn(xg, W1, b1, W2, b2, gw2, bexp)
    out = _run_combine(og, pos)
    return out.reshape(batch, seq, hidden)


# speedup vs baseline: 1.7738x; 1.0032x over previous
import functools

import jax
import jax.numpy as jnp
from jax import lax
from jax.experimental import pallas as pl
from jax.experimental.pallas import tpu as pltpu
from jax.experimental.pallas import tpu_sc as plsc

HIDDEN = 1024
FFN = 4096
E = 8
T = 4096
A = 2 * T
BLK = 256
P = A + E * BLK
NB = P // BLK
NBP = 48
TOK_BLK = 512

NTILE = 16
CH = A // NTILE
NG = CH // 16
PCOLS = 128
PROWS = P // PCOLS

GT = 32
GSL = P // GT
GCH = 40
GD = HIDDEN // 2
TPT = T // GT
CC = 16


def _router_body(x_ref, wr_ref, eidx_ref, ew_ref):
    xb = x_ref[...]
    logits = lax.dot_general(xb, wr_ref[...], (((1,), (1,)), ((), ())),
                             preferred_element_type=jnp.float32)
    m = jnp.max(logits, axis=1, keepdims=True)
    ex = jnp.exp(logits - m)
    probs = ex / jnp.sum(ex, axis=1, keepdims=True)

    iota = lax.broadcasted_iota(jnp.int32, probs.shape, 1)
    m1 = jnp.max(probs, axis=1, keepdims=True)
    idx1 = jnp.min(jnp.where(probs == m1, iota, E), axis=1, keepdims=True)
    p2 = jnp.where(iota == idx1, -jnp.inf, probs)
    m2 = jnp.max(p2, axis=1, keepdims=True)
    idx2 = jnp.min(jnp.where(p2 == m2, iota, E), axis=1, keepdims=True)
    eidx_ref[...] = jnp.concatenate([idx1.T, idx2.T], axis=0)
    ew_ref[...] = jnp.concatenate([m1.T, m2.T], axis=0)


def _dispatch_body(eidx_hbm, ew_hbm, gidx_hbm, gw_hbm, pos_hbm, bexp_hbm,
                   ids_v, ws_v, pos_v, gidx_v, gw_v, vec_v, all_v, cur_v,
                   bexp_v, rowi_v, sh_cnt, sh_gidx, sh_gw):
    wid = lax.axis_index("s")
    base = wid * CH
    lane = lax.iota(jnp.int32, 16)
    z16i = jnp.zeros((16,), jnp.int32)

    pltpu.sync_copy(eidx_hbm.at[pl.ds(base, CH)], ids_v)
    pltpu.sync_copy(ew_hbm.at[pl.ds(base, CH)], ws_v)

    def _zrow(i, c):
        for k in range(PCOLS // 16):
            gidx_v[i, pl.ds(k * 16, 16)] = z16i
            gw_v[i, pl.ds(k * 16, 16)] = z16i
        return c
    lax.fori_loop(0, PROWS, _zrow, 0)

    def _riota(j, c):
        rowi_v[pl.ds(j * 16, 16)] = j * 16 + lane
        return c
    lax.fori_loop(0, PROWS // 16, _riota, 0)

    def _hist(g, cnt):
        ids16 = ids_v[pl.ds(g * 16, 16)]
        for e in range(E):
            c = jnp.sum((ids16 == e).astype(jnp.int32))
            cnt = cnt + jnp.where(lane == e, c, 0)
        return cnt
    cnt = lax.fori_loop(0, NG, _hist, z16i)
    vec_v[...] = cnt
    pltpu.sync_copy(vec_v, sh_cnt.at[pl.ds(wid * 16, 16)])

    @pl.when(wid == 0)
    def _():
        pltpu.sync_copy(gidx_v, sh_gidx)
        pltpu.sync_copy(gw_v, sh_gw)

    plsc.subcore_barrier()

    pltpu.sync_copy(sh_cnt, all_v)
    tot = z16i
    pre = z16i
    for w in range(NTILE):
        row = all_v[pl.ds(w * 16, 16)]
        tot = tot + row
        pre = pre + jnp.where(w < wid, row, z16i)
    padded = ((tot + (BLK - 1)) >> 8) << 8
    inc = plsc.cumsum(padded)
    off = inc - padded
    cur_v[...] = off + pre

    @pl.when(wid == 0)
    def _():
        binc = plsc.cumsum(padded >> 8)
        nbu = jnp.sum(jnp.where(lane == E - 1, binc, 0))
        for c in range(NBP // 16):
            bv = lane + c * 16
            acc = z16i
            for e in range(E):
                s_e = jnp.sum(jnp.where(lane == e, binc, 0))
                acc = acc + (bv >= s_e).astype(jnp.int32)
            ch = jnp.minimum(acc, E - 1)
            if c * 16 <= NB < (c + 1) * 16:
                ch = jnp.where(lane == NB - c * 16, nbu, ch)
            bexp_v[pl.ds(c * 16, 16)] = ch
        pltpu.sync_copy(bexp_v, bexp_hbm)

    def _assign(g, c):
        ids16 = ids_v[pl.ds(g * 16, 16)]
        ws16 = ws_v[pl.ds(g * 16, 16)]
        tok16 = (base + g * 16 + lane) & (T - 1)
        curv = plsc.load_gather(cur_v, [ids16])
        rank = z16i
        upd = z16i
        for e in range(E):
            oh = ids16 == e
            ohi = oh.astype(jnp.int32)
            cs = plsc.cumsum(ohi)
            rank = rank + jnp.where(oh, cs - 1, z16i)
            upd = upd + jnp.where(lane == e, jnp.sum(ohi), 0)
        dest = curv + rank
        cur_v[...] = cur_v[...] + upd
        plsc.store_scatter(gidx_v, [dest >> 7, dest & (PCOLS - 1)], tok16)
        plsc.store_scatter(gw_v, [dest >> 7, dest & (PCOLS - 1)],
                           plsc.bitcast(ws16, jnp.int32))
        pos_v[pl.ds(g * 16, 16)] = dest
        return c
    lax.fori_loop(0, NG, _assign, 0)

    pltpu.sync_copy(pos_v, pos_hbm.at[pl.ds(base, CH)])

    plsc.subcore_barrier()
    pltpu.sync_copy(gidx_v, sh_gidx.at[rowi_v], add=True)
    pltpu.sync_copy(gw_v, sh_gw.at[rowi_v], add=True)
    plsc.subcore_barrier()

    @pl.when(wid < PROWS // 8)
    def _():
        pltpu.sync_copy(sh_gidx.at[pl.ds(wid * 8, 8)],
                        gidx_hbm.at[pl.ds(wid * 8, 8)])
        pltpu.sync_copy(sh_gw.at[pl.ds(wid * 8, 8)],
                        gw_hbm.at[pl.ds(wid * 8, 8)])


def _gather_body(flat_hbm, gidx_hbm, xg_hbm, idx_v, rows0_v, rows1_v,
                 rows2_v, rows3_v, rows4_v, sem0, sem1, sem2, sem3, sem4,
                 ssem0, ssem1, ssem2, ssem3, ssem4):
    wid = lax.axis_index("s") * 2 + lax.axis_index("c")
    base = wid * GSL
    nch = GSL // GCH
    nb_ = 5
    depth = 4
    bufs = (rows0_v, rows1_v, rows2_v, rows3_v, rows4_v)
    gsems = (sem0, sem1, sem2, sem3, sem4)
    ssems = (ssem0, ssem1, ssem2, ssem3, ssem4)
    pltpu.sync_copy(gidx_hbm.at[pl.ds(base, GSL)], idx_v)

    def _g(j):
        return pltpu.async_copy(
            flat_hbm.at[idx_v.at[pl.ds(j * GCH, GCH)]],
            bufs[j % nb_], gsems[j % nb_])

    gcp = [None] * nb_
    scp = [None] * nb_
    for k in range(min(depth, nch)):
        gcp[k] = _g(k)
    for j in range(nch):
        b = j % nb_
        nxt = j + depth
        if nxt < nch:
            nb2 = nxt % nb_
            if scp[nb2] is not None:
                scp[nb2].wait()
                scp[nb2] = None
            gcp[nb2] = _g(nxt)
        gcp[b].wait()
        scp[b] = pltpu.async_copy(
            bufs[b], xg_hbm.at[pl.ds(base + j * GCH, GCH)], ssems[b])
    for cp in scp:
        if cp is not None:
            cp.wait()


def _ffn1_body(bexp_ref, xg_ref, w1_ref, b1_ref, h_ref):
    @pl.when(pl.program_id(0) < bexp_ref[NB])
    def _():
        xi = xg_ref[...]
        xlo = lax.bitcast_convert_type(xi << 16, jnp.float32)
        xhi = lax.bitcast_convert_type(xi & jnp.int32(-65536), jnp.float32)
        w1 = w1_ref[0]
        h = lax.dot_general(xlo, w1[:, :GD], (((1,), (1,)), ((), ())),
                            preferred_element_type=jnp.float32)
        h = h + lax.dot_general(xhi, w1[:, GD:], (((1,), (1,)), ((), ())),
                                preferred_element_type=jnp.float32)
        h = h + b1_ref[0]
        h_ref[...] = h * jax.nn.sigmoid(h)


def _ffn2_body(bexp_ref, h_ref, w2_ref, b2_ref, gw_ref, og_ref):
    @pl.when(pl.program_id(0) < bexp_ref[NB])
    def _():
        o = lax.dot_general(h_ref[...], w2_ref[0], (((1,), (1,)), ((), ())),
                            preferred_element_type=jnp.float32)
        o = o + b2_ref[0]
        og_ref[...] = o * gw_ref[0, 0][:, None]


def _combine_body(og_hbm, pos_hbm, out_hbm, idx0_v, idx1_v, bufa0_v, bufa1_v,
                  bufb0_v, bufb1_v, sem0, sem1):
    wid = lax.axis_index("s") * 2 + lax.axis_index("c")
    tbase = wid * TPT
    nch = TPT // CC
    bufs = ((bufa0_v, bufa1_v), (bufb0_v, bufb1_v))
    sems = (sem0, sem1)
    pltpu.sync_copy(pos_hbm.at[pl.ds(tbase, TPT)], idx0_v)
    pltpu.sync_copy(pos_hbm.at[pl.ds(T + tbase, TPT)], idx1_v)

    def _start(j):
        b0, b1 = bufs[j % 2]
        s = sems[j % 2]
        c0 = pltpu.async_copy(og_hbm.at[idx0_v.at[pl.ds(j * CC, CC)]], b0, s)
        c1 = pltpu.async_copy(og_hbm.at[idx1_v.at[pl.ds(j * CC, CC)]], b1, s)
        return (c0, c1)

    cps = [None, None]
    cps[0] = _start(0)
    for j in range(nch):
        if j + 1 < nch:
            cps[(j + 1) % 2] = _start(j + 1)
        cps[j % 2][0].wait()
        cps[j % 2][1].wait()
        b0, b1 = bufs[j % 2]

        def _row(i, c):
            for k in range(HIDDEN // 16):
                s = pl.ds(k * 16, 16)
                b0[i, s] = b0[i, s] + b1[i, s]
            return c
        lax.fori_loop(0, CC, _row, 0)
        pltpu.sync_copy(b0, out_hbm.at[pl.ds(tbase + j * CC, CC)])


def _run_router(flat, Wr):
    return pl.pallas_call(
        _router_body,
        grid=(T // TOK_BLK,),
        in_specs=[
            pl.BlockSpec((TOK_BLK, HIDDEN), lambda t: (t, 0)),
            pl.BlockSpec((E, HIDDEN), lambda t: (0, 0)),
        ],
        out_specs=[
            pl.BlockSpec((2, TOK_BLK), lambda t: (0, t)),
            pl.BlockSpec((2, TOK_BLK), lambda t: (0, t)),
        ],
        out_shape=[
            jax.ShapeDtypeStruct((2, T), jnp.int32),
            jax.ShapeDtypeStruct((2, T), jnp.float32),
        ],
    )(flat, Wr)


def _run_dispatch(eidx, ew):
    mesh1 = plsc.VectorSubcoreMesh(core_axis_name="c", subcore_axis_name="s",
                                   num_cores=1, num_subcores=NTILE)
    dispatch = functools.partial(
        pl.kernel,
        out_type=[
            jax.ShapeDtypeStruct((PROWS, PCOLS), jnp.int32),
            jax.ShapeDtypeStruct((PROWS, PCOLS), jnp.int32),
            jax.ShapeDtypeStruct((A,), jnp.int32),
            jax.ShapeDtypeStruct((NBP,), jnp.int32),
        ],
        mesh=mesh1,
        scratch_types=[
            pltpu.VMEM((CH,), jnp.int32),
            pltpu.VMEM((CH,), jnp.float32),
            pltpu.VMEM((CH,), jnp.int32),
            pltpu.VMEM((PROWS, PCOLS), jnp.int32),
            pltpu.VMEM((PROWS, PCOLS), jnp.int32),
            pltpu.VMEM((16,), jnp.int32),
            pltpu.VMEM((NTILE * 16,), jnp.int32),
            pltpu.VMEM((16,), jnp.int32),
            pltpu.VMEM((NBP,), jnp.int32),
            pltpu.VMEM((PROWS,), jnp.int32),
            pltpu.VMEM_SHARED((NTILE * 16,), jnp.int32),
            pltpu.VMEM_SHARED((PROWS, PCOLS), jnp.int32),
            pltpu.VMEM_SHARED((PROWS, PCOLS), jnp.int32),
        ],
        compiler_params=pltpu.CompilerParams(needs_layout_passes=False),
    )(_dispatch_body)
    gidx2, gw2i, pos, bexp = dispatch(eidx.reshape(A), ew.reshape(A))
    gidx = gidx2.reshape(P)
    gw2 = lax.bitcast_convert_type(gw2i, jnp.float32)
    return gidx, gw2, pos, bexp


def _run_gather(flat, gidx):
    mesh2 = plsc.VectorSubcoreMesh(core_axis_name="c", subcore_axis_name="s",
                                   num_cores=2, num_subcores=NTILE)
    gather = functools.partial(
        pl.kernel,
        out_type=jax.ShapeDtypeStruct((P, GD), jnp.int32),
        mesh=mesh2,
        scratch_types=[
            pltpu.VMEM((GSL,), jnp.int32),
            pltpu.VMEM((GCH, GD), jnp.int32),
            pltpu.VMEM((GCH, GD), jnp.int32),
            pltpu.VMEM((GCH, GD), jnp.int32),
            pltpu.VMEM((GCH, GD), jnp.int32),
            pltpu.VMEM((GCH, GD), jnp.int32),
            pltpu.SemaphoreType.DMA,
            pltpu.SemaphoreType.DMA,
            pltpu.SemaphoreType.DMA,
            pltpu.SemaphoreType.DMA,
            pltpu.SemaphoreType.DMA,
            pltpu.SemaphoreType.DMA,
            pltpu.SemaphoreType.DMA,
            pltpu.SemaphoreType.DMA,
            pltpu.SemaphoreType.DMA,
            pltpu.SemaphoreType.DMA,
        ],
        compiler_params=pltpu.CompilerParams(needs_layout_passes=False),
    )(_gather_body)
    return gather(flat, gidx)


def _run_ffn(xg, W1, b1, W2, b2, gw2, bexp):
    h_all = pl.pallas_call(
        _ffn1_body,
        grid_spec=pltpu.PrefetchScalarGridSpec(
            num_scalar_prefetch=1,
            grid=(NB,),
            in_specs=[
                pl.BlockSpec((BLK, GD), lambda b, be: (b, 0)),
                pl.BlockSpec((1, FFN, HIDDEN), lambda b, be: (be[b], 0, 0)),
                pl.BlockSpec((1, 1, FFN), lambda b, be: (be[b], 0, 0)),
            ],
            out_specs=pl.BlockSpec((BLK, FFN), lambda b, be: (b, 0)),
        ),
        out_shape=jax.ShapeDtypeStruct((P, FFN), jnp.float32),
    )(bexp, xg, W1, b1.reshape(E, 1, FFN))

    og = pl.pallas_call(
        _ffn2_body,
        grid_spec=pltpu.PrefetchScalarGridSpec(
            num_scalar_prefetch=1,
            grid=(NB,),
            in_specs=[
                pl.BlockSpec((BLK, FFN), lambda b, be: (b, 0)),
                pl.BlockSpec((1, HIDDEN, FFN), lambda b, be: (be[b], 0, 0)),
                pl.BlockSpec((1, 1, HIDDEN), lambda b, be: (be[b], 0, 0)),
                pl.BlockSpec((1, 1, BLK), lambda b, be: (b, 0, 0)),
            ],
            out_specs=pl.BlockSpec((BLK, HIDDEN), lambda b, be: (b, 0)),
        ),
        out_shape=jax.ShapeDtypeStruct((P, HIDDEN), jnp.float32),
    )(bexp, h_all, W2, b2.reshape(E, 1, HIDDEN), gw2.reshape(NB, 1, BLK))
    return og


def _run_combine(og, pos):
    mesh2 = plsc.VectorSubcoreMesh(core_axis_name="c", subcore_axis_name="s",
                                   num_cores=2, num_subcores=NTILE)
    combine = functools.partial(
        pl.kernel,
        out_type=jax.ShapeDtypeStruct((T, HIDDEN), jnp.float32),
        mesh=mesh2,
        scratch_types=[
            pltpu.VMEM((TPT,), jnp.int32),
            pltpu.VMEM((TPT,), jnp.int32),
            pltpu.VMEM((CC, HIDDEN), jnp.float32),
            pltpu.VMEM((CC, HIDDEN), jnp.float32),
            pltpu.VMEM((CC, HIDDEN), jnp.float32),
            pltpu.VMEM((CC, HIDDEN), jnp.float32),
            pltpu.SemaphoreType.DMA,
            pltpu.SemaphoreType.DMA,
        ],
        compiler_params=pltpu.CompilerParams(needs_layout_passes=False),
    )(_combine_body)
    return combine(og, pos)


def kernel(x, Wr, W1, b1, W2, b2):
    batch, seq, hidden = x.shape
    flat = x.reshape(T, hidden)
    eidx, ew = _run_router(flat, Wr)
    gidx, gw2, pos, bexp = _run_dispatch(eidx, ew)
    flatb = flat.astype(jnp.bfloat16)
    flat_pack = lax.bitcast_convert_type(
        jnp.stack([flatb[:, :GD], flatb[:, GD:]], axis=-1), jnp.int32)
    xg = _run_gather(flat_pack, gidx)
    og = _run_ffn(xg, W1, b1, W2, b2, gw2, bexp)
    out = _run_combine(og, pos)
    return out.reshape(batch, seq, hidden)
